# baseline (speedup 1.0000x reference)
import functools

import jax
import jax.numpy as jnp
from jax import lax
from jax.experimental import pallas as pl
from jax.experimental.pallas import tpu as pltpu
from jax.experimental.pallas import tpu_sc as plsc

N = 10000
E = 160000
LATENT = 128
NUM_TYPES = 9

NC = 2
NS = 16
NW = NC * NS
CHUNK = 128
E_PAD = 163840
ROWS_W = E_PAD // NW // CHUNK
EW = E_PAD // NW
ACC_ROWS = 10240
ROWS_T = ACC_ROWS // NS

@functools.cache
def _mesh():
  return plsc.VectorSubcoreMesh(
      core_axis_name="c", subcore_axis_name="s", num_cores=NC,
      num_subcores=NS)

f32 = jnp.float32
i32 = jnp.int32
bf16 = jnp.bfloat16


_NSLOT = 4
_TROWS = N // NS


def _sc_gather_sum(Ps, Pd, srcs, dsts):
  @functools.partial(
      pl.kernel,
      out_type=jax.ShapeDtypeStruct((E_PAD, LATENT), bf16),
      mesh=_mesh(),
      scratch_types=[
          pltpu.VMEM((ROWS_W, CHUNK), i32),
          pltpu.VMEM((ROWS_W, CHUNK), i32),
      ] + [pltpu.VMEM((CHUNK, LATENT), bf16)] * (2 * _NSLOT)
        + [pltpu.SemaphoreType.DMA] * (3 * _NSLOT),
      compiler_params=pltpu.CompilerParams(use_tc_tiling_on_sc=False),
  )
  def k(ps_hbm, pd_hbm, srcs_hbm, dsts_hbm, out_hbm, idx_s, idx_d,
        *rest):
    bufs_s = rest[0:_NSLOT]
    bufs_d = rest[_NSLOT:2 * _NSLOT]
    sems_s = rest[2 * _NSLOT:3 * _NSLOT]
    sems_d = rest[3 * _NSLOT:4 * _NSLOT]
    sems_w = rest[4 * _NSLOT:5 * _NSLOT]
    c = lax.axis_index("c")
    s = lax.axis_index("s")
    w = c * NS + s
    r0 = w * ROWS_W
    pltpu.sync_copy(srcs_hbm.at[pl.ds(r0, ROWS_W)], idx_s)
    pltpu.sync_copy(dsts_hbm.at[pl.ds(r0, ROWS_W)], idx_d)

    def g_issue(j, b):
      pltpu.async_copy(ps_hbm.at[idx_s.at[j]], bufs_s[b], sems_s[b])
      pltpu.async_copy(pd_hbm.at[idx_d.at[j]], bufs_d[b], sems_d[b])

    def g_wait(j, b):
      pltpu.make_async_copy(ps_hbm.at[idx_s.at[j]], bufs_s[b],
                            sems_s[b]).wait()
      pltpu.make_async_copy(pd_hbm.at[idx_d.at[j]], bufs_d[b],
                            sems_d[b]).wait()

    def out_slice(j):
      return out_hbm.at[pl.ds(w * EW + j * CHUNK, CHUNK)]

    def add_and_store(j, b):
      bs, bd = bufs_s[b], bufs_d[b]

      def add_row(r, c2):
        for q in range(LATENT // 32):
          sl = pl.ds(q * 32, 32)
          bs[r, sl] = bs[r, sl] + bd[r, sl]
        return c2

      lax.fori_loop(0, CHUNK, add_row, 0)
      pltpu.async_copy(bs, out_slice(j), sems_w[b])

    def w_drain(j, b):
      pltpu.make_async_copy(bufs_s[b], out_slice(j), sems_w[b]).wait()

    for b in range(_NSLOT):
      g_issue(b, b)

    def group_body(t, carry):
      for b in range(_NSLOT):
        j = _NSLOT * t + b
        g_wait(j, b)
        add_and_store(j, b)
        w_drain(j, b)
        pl.when(t < ROWS_W // _NSLOT - 1)(
            functools.partial(g_issue, j + _NSLOT, b))
      return carry

    lax.fori_loop(0, ROWS_W // _NSLOT, group_body, 0)

  return k(Ps, Pd, srcs, dsts)


def _sc_feature_diff(T, srcs, dsts):
  D = 16

  @functools.partial(
      pl.kernel,
      out_type=jax.ShapeDtypeStruct((E_PAD, D), f32),
      mesh=_mesh(),
      scratch_types=[
          pltpu.VMEM((ROWS_W, CHUNK), i32),
          pltpu.VMEM((ROWS_W, CHUNK), i32),
          pltpu.VMEM((CHUNK, D), f32),
          pltpu.VMEM((CHUNK, D), f32),
          pltpu.VMEM((CHUNK, D), f32),
          pltpu.VMEM((CHUNK, D), f32),
          pltpu.SemaphoreType.DMA,
          pltpu.SemaphoreType.DMA,
          pltpu.SemaphoreType.DMA,
          pltpu.SemaphoreType.DMA,
          pltpu.SemaphoreType.DMA,
          pltpu.SemaphoreType.DMA,
      ],
      compiler_params=pltpu.CompilerParams(use_tc_tiling_on_sc=False),
  )
  def k(t_hbm, srcs_hbm, dsts_hbm, out_hbm, idx_s, idx_d,
        bs0, bd0, bs1, bd1, ss0, sd0, ss1, sd1, sw0, sw1):
    w = lax.axis_index("c") * NS + lax.axis_index("s")
    r0 = w * ROWS_W
    pltpu.sync_copy(srcs_hbm.at[pl.ds(r0, ROWS_W)], idx_s)
    pltpu.sync_copy(dsts_hbm.at[pl.ds(r0, ROWS_W)], idx_d)

    bufs = ((bs0, bd0, ss0, sd0, sw0), (bs1, bd1, ss1, sd1, sw1))

    def g_issue(j, slot):
      bs, bd, ss, sd, _ = bufs[slot]
      pltpu.async_copy(t_hbm.at[idx_s.at[j]], bs, ss)
      pltpu.async_copy(t_hbm.at[idx_d.at[j]], bd, sd)

    def g_wait(j, slot):
      bs, bd, ss, sd, _ = bufs[slot]
      pltpu.make_async_copy(t_hbm.at[idx_s.at[j]], bs, ss).wait()
      pltpu.make_async_copy(t_hbm.at[idx_d.at[j]], bd, sd).wait()

    def out_slice(j):
      return out_hbm.at[pl.ds(w * EW + j * CHUNK, CHUNK)]

    def sub_and_store(j, slot):
      bs, bd, _, _, sw = bufs[slot]

      def sub_row(r, c2):
        bs[r, pl.ds(0, 16)] = bs[r, pl.ds(0, 16)] - bd[r, pl.ds(0, 16)]
        return c2

      lax.fori_loop(0, CHUNK, sub_row, 0)
      pltpu.async_copy(bs, out_slice(j), sw)

    def w_drain(j, slot):
      bs, _, _, _, sw = bufs[slot]
      pltpu.make_async_copy(bs, out_slice(j), sw).wait()

    g_issue(0, 0)

    def pair_body(t, carry):
      j0 = 2 * t
      j1 = j0 + 1
      pl.when(t > 0)(lambda: w_drain(j1 - 2, 1))
      g_issue(j1, 1)
      g_wait(j0, 0)
      sub_and_store(j0, 0)
      w_drain(j0, 0)
      pl.when(t < ROWS_W // 2 - 1)(lambda: g_issue(j0 + 2, 0))
      g_wait(j1, 1)
      sub_and_store(j1, 1)
      return carry

    lax.fori_loop(0, ROWS_W // 2, pair_body, 0)
    w_drain(ROWS_W - 1, 1)

  return k(T, srcs, dsts)


def _sc_segment_sum(e_new, dsts):
  @functools.partial(
      pl.kernel,
      out_type=jax.ShapeDtypeStruct((NC, ACC_ROWS, LATENT), bf16),
      mesh=_mesh(),
      scratch_types=[
          pltpu.VMEM((ROWS_W, CHUNK), i32),
          pltpu.VMEM((CHUNK, LATENT), bf16),
          pltpu.VMEM((CHUNK, LATENT), bf16),
          pltpu.VMEM_SHARED((ACC_ROWS, LATENT), bf16),
          pltpu.SemaphoreType.DMA,
          pltpu.SemaphoreType.DMA,
      ],
      compiler_params=pltpu.CompilerParams(use_tc_tiling_on_sc=False),
  )
  def k(e_hbm, dsts_hbm, out_hbm, idx_d, eb0, eb1, acc, sr0, sr1):
    c = lax.axis_index("c")
    s = lax.axis_index("s")
    w = c * NS + s
    pltpu.sync_copy(dsts_hbm.at[pl.ds(w * ROWS_W, ROWS_W)], idx_d)

    bufs = ((eb0, sr0), (eb1, sr1))

    def e_slice(j):
      return e_hbm.at[pl.ds(w * EW + j * CHUNK, CHUNK)]

    def r_issue(j, slot):
      eb, sr = bufs[slot]
      pltpu.async_copy(e_slice(j), eb, sr)

    def r_wait(j, slot):
      eb, sr = bufs[slot]
      pltpu.make_async_copy(e_slice(j), eb, sr).wait()

    def zero_row(r, carry):
      for q in range(LATENT // 32):
        eb0[r, pl.ds(q * 32, 32)] = jnp.zeros((32,), bf16)
      return carry

    lax.fori_loop(0, CHUNK, zero_row, 0)
    for q in range(ROWS_T // CHUNK):
      pltpu.sync_copy(eb0, acc.at[pl.ds(s * ROWS_T + q * CHUNK, CHUNK)])
    plsc.subcore_barrier()

    r_issue(0, 0)

    def pair_body(t, carry):
      j0 = 2 * t
      j1 = j0 + 1
      r_issue(j1, 1)
      r_wait(j0, 0)
      pltpu.sync_copy(eb0, acc.at[idx_d.at[j0]], add=True)
      pl.when(t < ROWS_W // 2 - 1)(lambda: r_issue(j0 + 2, 0))
      r_wait(j1, 1)
      pltpu.sync_copy(eb1, acc.at[idx_d.at[j1]], add=True)
      return carry

    lax.fori_loop(0, ROWS_W // 2, pair_body, 0)
    plsc.subcore_barrier()

    for q in range(ROWS_T // CHUNK):
      r = s * ROWS_T + q * CHUNK
      pltpu.sync_copy(acc.at[pl.ds(r, CHUNK)], out_hbm.at[c, pl.ds(r, CHUNK)])

  return k(e_new, dsts)


BE = 2048
BN = 2000


def _ln(h, g, b):
  mu = jnp.mean(h, axis=-1, keepdims=True)
  xc = h - mu
  var = jnp.mean(xc * xc, axis=-1, keepdims=True)
  return xc * lax.rsqrt(var + 1e-5) * g + b


def _dot(a, b):
  return jnp.dot(a, b, preferred_element_type=f32)


def _full(shape):
  return pl.BlockSpec(shape, lambda i: tuple(0 for _ in shape))


def _tc_edge_encoder(diff, W0, W1, W2, consts):
  def body(d_ref, w0_ref, w1_ref, w2_ref, c_ref, out_ref):
    d = d_ref[...]
    rw = d[:, 0:3]
    rm = d[:, 3:6]
    nw = jnp.sqrt(jnp.sum(rw * rw, axis=-1, keepdims=True) + 1e-12)
    nm = jnp.sqrt(jnp.sum(rm * rm, axis=-1, keepdims=True) + 1e-12)
    feat = jnp.concatenate([rw, nw, rm, nm], axis=-1)
    h0 = jnp.maximum(_dot(feat, w0_ref[...]) + c_ref[0], 0.0)
    h1 = jnp.maximum(_dot(h0, w1_ref[...]) + c_ref[1], 0.0)
    h2 = _dot(h1, w2_ref[...]) + c_ref[2]
    out_ref[...] = _ln(h2, c_ref[3], c_ref[4])

  return pl.pallas_call(
      body,
      grid=(E_PAD // BE,),
      in_specs=[
          pl.BlockSpec((BE, 16), lambda i: (i, 0)),
          _full((8, LATENT)),
          _full((LATENT, LATENT)),
          _full((LATENT, LATENT)),
          _full((8, LATENT)),
      ],
      out_specs=pl.BlockSpec((BE, LATENT), lambda i: (i, 0)),
      out_shape=jax.ShapeDtypeStruct((E_PAD, LATENT), f32),
  )(diff, W0, W1, W2, consts)


def _tc_node_encoder(wp, pwp, tcol, W0, W1, W2, consts, Wp):
  def body(wp_ref, pwp_ref, t_ref, w0_ref, w1_ref, w2_ref, c_ref, wp_proj_ref,
           nlat_ref, ps_ref, pd_ref):
    vel = wp_ref[...] - pwp_ref[...]
    t = t_ref[...].astype(i32)
    oh = jnp.where(
        t == lax.broadcasted_iota(i32, (BN, NUM_TYPES), 1), 1.0, 0.0)
    feat = jnp.concatenate([vel, oh], axis=-1)
    h0 = jnp.maximum(_dot(feat, w0_ref[...]) + c_ref[0], 0.0)
    h1 = jnp.maximum(_dot(h0, w1_ref[...]) + c_ref[1], 0.0)
    h2 = _dot(h1, w2_ref[...]) + c_ref[2]
    nl = _ln(h2, c_ref[3], c_ref[4])
    nlat_ref[...] = nl
    proj = _dot(nl, wp_proj_ref[...])
    ps_ref[...] = proj[:, :LATENT].astype(bf16)
    pd_ref[...] = proj[:, LATENT:].astype(bf16)

  return pl.pallas_call(
      body,
      grid=(N // BN,),
      in_specs=[
          pl.BlockSpec((BN, 3), lambda i: (i, 0)),
          pl.BlockSpec((BN, 3), lambda i: (i, 0)),
          pl.BlockSpec((BN, 1), lambda i: (i, 0)),
          _full((NUM_TYPES + 3, LATENT)),
          _full((LATENT, LATENT)),
          _full((LATENT, LATENT)),
          _full((8, LATENT)),
          _full((LATENT, 2 * LATENT)),
      ],
      out_specs=[
          pl.BlockSpec((BN, LATENT), lambda i: (i, 0)),
          pl.BlockSpec((BN, LATENT), lambda i: (i, 0)),
          pl.BlockSpec((BN, LATENT), lambda i: (i, 0)),
      ],
      out_shape=[jax.ShapeDtypeStruct((N, LATENT), f32),
                 jax.ShapeDtypeStruct((N, LATENT), bf16),
                 jax.ShapeDtypeStruct((N, LATENT), bf16)],
  )(wp, pwp, tcol, W0, W1, W2, consts, Wp)


def _tc_edge_block(gsum, elat, W0e, W1, W2, consts):
  def body(g_ref, e_ref, w0_ref, w1_ref, w2_ref, c_ref, en_ref, eo_ref):
    e = e_ref[...]
    h0 = jnp.maximum(
        g_ref[...].astype(f32) + _dot(e, w0_ref[...]) + c_ref[0], 0.0)
    h1 = jnp.maximum(_dot(h0, w1_ref[...]) + c_ref[1], 0.0)
    h2 = _dot(h1, w2_ref[...]) + c_ref[2]
    y = _ln(h2, c_ref[3], c_ref[4])
    en_ref[...] = y.astype(bf16)
    eo_ref[...] = e + y

  return pl.pallas_call(
      body,
      grid=(E_PAD // BE,),
      in_specs=[
          pl.BlockSpec((BE, LATENT), lambda i: (i, 0)),
          pl.BlockSpec((BE, LATENT), lambda i: (i, 0)),
          _full((LATENT, LATENT)),
          _full((LATENT, LATENT)),
          _full((LATENT, LATENT)),
          _full((8, LATENT)),
      ],
      out_specs=[
          pl.BlockSpec((BE, LATENT), lambda i: (i, 0)),
          pl.BlockSpec((BE, LATENT), lambda i: (i, 0)),
      ],
      out_shape=[jax.ShapeDtypeStruct((E_PAD, LATENT), bf16),
                 jax.ShapeDtypeStruct((E_PAD, LATENT), f32)],
  )(gsum, elat, W0e, W1, W2, consts)


def _tc_node_block(nlat, agg2, W0, W1, W2, consts, Wp):
  def body(n_ref, a_ref, w0_ref, w1_ref, w2_ref, c_ref, wp_ref,
           no_ref, ps_ref, pd_ref):
    nl = n_ref[...]
    agg = a_ref[0].astype(f32) + a_ref[1].astype(f32)
    x = jnp.concatenate([nl, agg], axis=-1)
    h0 = jnp.maximum(_dot(x, w0_ref[...]) + c_ref[0], 0.0)
    h1 = jnp.maximum(_dot(h0, w1_ref[...]) + c_ref[1], 0.0)
    h2 = _dot(h1, w2_ref[...]) + c_ref[2]
    nl_new = nl + _ln(h2, c_ref[3], c_ref[4])
    no_ref[...] = nl_new
    proj = _dot(nl_new, wp_ref[...])
    ps_ref[...] = proj[:, :LATENT].astype(bf16)
    pd_ref[...] = proj[:, LATENT:].astype(bf16)

  return pl.pallas_call(
      body,
      grid=(N // BN,),
      in_specs=[
          pl.BlockSpec((BN, LATENT), lambda i: (i, 0)),
          pl.BlockSpec((NC, BN, LATENT), lambda i: (0, i, 0)),
          _full((2 * LATENT, LATENT)),
          _full((LATENT, LATENT)),
          _full((LATENT, LATENT)),
          _full((8, LATENT)),
          _full((LATENT, 2 * LATENT)),
      ],
      out_specs=[
          pl.BlockSpec((BN, LATENT), lambda i: (i, 0)),
          pl.BlockSpec((BN, LATENT), lambda i: (i, 0)),
          pl.BlockSpec((BN, LATENT), lambda i: (i, 0)),
      ],
      out_shape=[jax.ShapeDtypeStruct((N, LATENT), f32),
                 jax.ShapeDtypeStruct((N, LATENT), bf16),
                 jax.ShapeDtypeStruct((N, LATENT), bf16)],
  )(nlat, agg2, W0, W1, W2, consts, Wp)


def _tc_node_block_last(nlat, agg2, W0, W1, W2, consts):
  def body(n_ref, a_ref, w0_ref, w1_ref, w2_ref, c_ref, no_ref):
    nl = n_ref[...]
    agg = a_ref[0].astype(f32) + a_ref[1].astype(f32)
    x = jnp.concatenate([nl, agg], axis=-1)
    h0 = jnp.maximum(_dot(x, w0_ref[...]) + c_ref[0], 0.0)
    h1 = jnp.maximum(_dot(h0, w1_ref[...]) + c_ref[1], 0.0)
    h2 = _dot(h1, w2_ref[...]) + c_ref[2]
    no_ref[...] = nl + _ln(h2, c_ref[3], c_ref[4])

  return pl.pallas_call(
      body,
      grid=(N // BN,),
      in_specs=[
          pl.BlockSpec((BN, LATENT), lambda i: (i, 0)),
          pl.BlockSpec((NC, BN, LATENT), lambda i: (0, i, 0)),
          _full((2 * LATENT, LATENT)),
          _full((LATENT, LATENT)),
          _full((LATENT, LATENT)),
          _full((8, LATENT)),
      ],
      out_specs=pl.BlockSpec((BN, LATENT), lambda i: (i, 0)),
      out_shape=jax.ShapeDtypeStruct((N, LATENT), f32),
  )(nlat, agg2, W0, W1, W2, consts)


def _tc_decoder(nlat, wp, pwp, tcol, W0, W1, W2p, consts):
  def body(n_ref, wp_ref, pwp_ref, t_ref, w0_ref, w1_ref, w2_ref, c_ref,
           out_ref):
    h0 = jnp.maximum(_dot(n_ref[...], w0_ref[...]) + c_ref[0], 0.0)
    h1 = jnp.maximum(_dot(h0, w1_ref[...]) + c_ref[1], 0.0)
    h2 = _dot(h1, w2_ref[...]) + c_ref[2]
    acc = h2 * c_ref[3] + c_ref[4]
    wpv = wp_ref[...]
    pred_pos = 2.0 * wpv + acc[:, 0:3] - pwp_ref[...]
    mask = t_ref[...] == 0.0
    out_ref[...] = jnp.where(mask, pred_pos, wpv)

  return pl.pallas_call(
      body,
      grid=(N // BN,),
      in_specs=[
          pl.BlockSpec((BN, LATENT), lambda i: (i, 0)),
          pl.BlockSpec((BN, 3), lambda i: (i, 0)),
          pl.BlockSpec((BN, 3), lambda i: (i, 0)),
          pl.BlockSpec((BN, 1), lambda i: (i, 0)),
          _full((LATENT, LATENT)),
          _full((LATENT, LATENT)),
          _full((LATENT, LATENT)),
          _full((8, LATENT)),
      ],
      out_specs=pl.BlockSpec((BN, 3), lambda i: (i, 0)),
      out_shape=jax.ShapeDtypeStruct((N, 3), f32),
  )(nlat, wp, pwp, tcol, W0, W1, W2p, consts)


def _pack_consts(b0, b1, b2, g=None, b=None):
  rows = [b0, b1, b2]
  rows.append(g if g is not None else jnp.zeros((LATENT,), f32))
  rows.append(b if b is not None else jnp.zeros((LATENT,), f32))
  rows += [jnp.zeros((LATENT,), f32)] * 3
  return jnp.stack([jnp.pad(r, (0, LATENT - r.shape[0])) for r in rows])


def kernel(world_pos, prev_world_pos, mesh_pos, params, node_type, edge_index):
  src = edge_index[0].astype(i32)
  dst = edge_index[1].astype(i32)
  pad = E_PAD - E
  src_g = jnp.concatenate([src, jnp.zeros((pad,), i32)]).reshape(-1, CHUNK)
  dst_g = jnp.concatenate([dst, jnp.zeros((pad,), i32)]).reshape(-1, CHUNK)
  dst_s = jnp.concatenate([dst, jnp.full((pad,), N, i32)]).reshape(-1, CHUNK)

  T = jnp.concatenate(
      [world_pos, mesh_pos, jnp.zeros((N, 10), f32)], axis=1)
  tcol = node_type.astype(f32)[:, None]

  p = params

  def fold_first(mlp, mean, std):
    w0 = mlp['W0'] / std[:, None]
    b0 = mlp['b0'] - jnp.dot(mean / std, mlp['W0'])
    return w0, b0

  ew0, eb0 = fold_first(p['enc_edge'], p['mesh_norm']['mean'],
                        p['mesh_norm']['std'])
  nw0, nb0 = fold_first(p['enc_node'], p['node_norm']['mean'],
                        p['node_norm']['std'])
  enc_e = p['enc_edge']
  enc_n = p['enc_node']

  diff = _sc_feature_diff(T, src_g, dst_g)
  elat = _tc_edge_encoder(
      diff, ew0, enc_e['W1'], enc_e['W2'],
      _pack_consts(eb0, enc_e['b1'], enc_e['b2'], enc_e['ln_g'],
                   enc_e['ln_b']))

  def proj_weights(blk):
    w0 = blk['edge_mlp']['W0']
    return jnp.concatenate([w0[:LATENT], w0[LATENT:2 * LATENT]], axis=1)

  nlat, Ps, Pd = _tc_node_encoder(
      world_pos, prev_world_pos, tcol, nw0, enc_n['W1'], enc_n['W2'],
      _pack_consts(nb0, enc_n['b1'], enc_n['b2'], enc_n['ln_g'],
                   enc_n['ln_b']),
      proj_weights(p['blocks'][0]))

  for b in range(len(p['blocks'])):
    blk = p['blocks'][b]
    em = blk['edge_mlp']
    nm = blk['node_mlp']
    gsum = _sc_gather_sum(Ps, Pd, src_g, dst_g)
    e_new, elat = _tc_edge_block(
        gsum, elat, em['W0'][2 * LATENT:], em['W1'], em['W2'],
        _pack_consts(em['b0'], em['b1'], em['b2'], em['ln_g'], em['ln_b']))
    agg2 = _sc_segment_sum(e_new, dst_s)
    nconsts = _pack_consts(nm['b0'], nm['b1'], nm['b2'], nm['ln_g'],
                           nm['ln_b'])
    if b + 1 < len(p['blocks']):
      nlat, Ps, Pd = _tc_node_block(
          nlat, agg2, nm['W0'], nm['W1'], nm['W2'], nconsts,
          proj_weights(p['blocks'][b + 1]))
    else:
      nlat = _tc_node_block_last(
          nlat, agg2, nm['W0'], nm['W1'], nm['W2'], nconsts)

  dec = p['dec']
  W2p = jnp.pad(dec['W2'], ((0, 0), (0, LATENT - 3)))
  dconsts = _pack_consts(
      dec['b0'], dec['b1'], dec['b2'],
      jnp.pad(p['out_norm']['std'], (0, LATENT - 3), constant_values=1.0),
      jnp.pad(p['out_norm']['mean'], (0, LATENT - 3)))
  return _tc_decoder(nlat, world_pos, prev_world_pos, tcol,
                     dec['W0'], dec['W1'], W2p, dconsts)


# speedup vs baseline: 1.3429x; 1.3429x over previous
import functools

import jax
import jax.numpy as jnp
from jax import lax
from jax.experimental import pallas as pl
from jax.experimental.pallas import tpu as pltpu
from jax.experimental.pallas import tpu_sc as plsc

N = 10000
E = 160000
LATENT = 128
NUM_TYPES = 9

NC = 2
NS = 16
NW = NC * NS
CHUNK = 128
E_PAD = 163840
ROWS_W = E_PAD // NW // CHUNK
EW = E_PAD // NW
ACC_ROWS = 10240
ROWS_T = ACC_ROWS // NS

@functools.cache
def _mesh():
  return plsc.VectorSubcoreMesh(
      core_axis_name="c", subcore_axis_name="s", num_cores=NC,
      num_subcores=NS)

f32 = jnp.float32
i32 = jnp.int32


def _sc_gather_sum(Ps, Pd, srcs, dsts):
  @functools.partial(
      pl.kernel,
      out_type=jax.ShapeDtypeStruct((E_PAD, LATENT), f32),
      mesh=_mesh(),
      scratch_types=[
          pltpu.VMEM((ROWS_W, CHUNK), i32),
          pltpu.VMEM((ROWS_W, CHUNK), i32),
      ] + [pltpu.VMEM((CHUNK, LATENT), f32)] * 6
        + [pltpu.SemaphoreType.DMA] * 6,
  )
  def k(ps_hbm, pd_hbm, srcs_hbm, dsts_hbm, out_hbm, idx_s, idx_d,
        bs0, bd0, ob0, bs1, bd1, ob1, ss0, sd0, sw0, ss1, sd1, sw1):
    w = lax.axis_index("c") * NS + lax.axis_index("s")
    r0 = w * ROWS_W
    pltpu.sync_copy(srcs_hbm.at[pl.ds(r0, ROWS_W)], idx_s)
    pltpu.sync_copy(dsts_hbm.at[pl.ds(r0, ROWS_W)], idx_d)

    bufs = ((bs0, bd0, ob0, ss0, sd0, sw0), (bs1, bd1, ob1, ss1, sd1, sw1))

    def g_issue(j, slot):
      bs, bd, _, ss, sd, _ = bufs[slot]
      pltpu.async_copy(ps_hbm.at[idx_s.at[j]], bs, ss)
      pltpu.async_copy(pd_hbm.at[idx_d.at[j]], bd, sd)

    def g_wait(j, slot):
      bs, bd, _, ss, sd, _ = bufs[slot]
      pltpu.make_async_copy(ps_hbm.at[idx_s.at[j]], bs, ss).wait()
      pltpu.make_async_copy(pd_hbm.at[idx_d.at[j]], bd, sd).wait()

    def out_slice(j):
      return out_hbm.at[pl.ds(w * EW + j * CHUNK, CHUNK)]

    def add_and_store(j, slot):
      bs, bd, ob, _, _, sw = bufs[slot]

      def add_row(r, c2):
        for q in range(LATENT // 16):
          sl = pl.ds(q * 16, 16)
          ob[r, sl] = bs[r, sl] + bd[r, sl]
        return c2

      lax.fori_loop(0, CHUNK, add_row, 0)
      pltpu.async_copy(ob, out_slice(j), sw)

    def w_drain(j, slot):
      _, _, ob, _, _, sw = bufs[slot]
      pltpu.make_async_copy(ob, out_slice(j), sw).wait()

    g_issue(0, 0)
    g_issue(1, 1)
    last_t = ROWS_W // 2 - 1

    def pair_body(t, carry):
      for slot in range(2):
        j = 2 * t + slot
        g_wait(j, slot)
        pl.when(t > 0)(functools.partial(w_drain, j - 2, slot))
        add_and_store(j, slot)
        pl.when(t < last_t)(functools.partial(g_issue, j + 2, slot))
      return carry

    lax.fori_loop(0, ROWS_W // 2, pair_body, 0)
    w_drain(ROWS_W - 2, 0)
    w_drain(ROWS_W - 1, 1)

  return k(Ps, Pd, srcs, dsts)


def _sc_feature_diff(T, srcs, dsts):
  D = 16

  @functools.partial(
      pl.kernel,
      out_type=jax.ShapeDtypeStruct((E_PAD, D), f32),
      mesh=_mesh(),
      scratch_types=[
          pltpu.VMEM((ROWS_W, CHUNK), i32),
          pltpu.VMEM((ROWS_W, CHUNK), i32),
          pltpu.VMEM((CHUNK, D), f32),
          pltpu.VMEM((CHUNK, D), f32),
          pltpu.VMEM((CHUNK, D), f32),
          pltpu.VMEM((CHUNK, D), f32),
          pltpu.SemaphoreType.DMA,
          pltpu.SemaphoreType.DMA,
          pltpu.SemaphoreType.DMA,
          pltpu.SemaphoreType.DMA,
          pltpu.SemaphoreType.DMA,
          pltpu.SemaphoreType.DMA,
      ],
      compiler_params=pltpu.CompilerParams(use_tc_tiling_on_sc=False),
  )
  def k(t_hbm, srcs_hbm, dsts_hbm, out_hbm, idx_s, idx_d,
        bs0, bd0, bs1, bd1, ss0, sd0, ss1, sd1, sw0, sw1):
    w = lax.axis_index("c") * NS + lax.axis_index("s")
    r0 = w * ROWS_W
    pltpu.sync_copy(srcs_hbm.at[pl.ds(r0, ROWS_W)], idx_s)
    pltpu.sync_copy(dsts_hbm.at[pl.ds(r0, ROWS_W)], idx_d)

    bufs = ((bs0, bd0, ss0, sd0, sw0), (bs1, bd1, ss1, sd1, sw1))

    def g_issue(j, slot):
      bs, bd, ss, sd, _ = bufs[slot]
      pltpu.async_copy(t_hbm.at[idx_s.at[j]], bs, ss)
      pltpu.async_copy(t_hbm.at[idx_d.at[j]], bd, sd)

    def g_wait(j, slot):
      bs, bd, ss, sd, _ = bufs[slot]
      pltpu.make_async_copy(t_hbm.at[idx_s.at[j]], bs, ss).wait()
      pltpu.make_async_copy(t_hbm.at[idx_d.at[j]], bd, sd).wait()

    def out_slice(j):
      return out_hbm.at[pl.ds(w * EW + j * CHUNK, CHUNK)]

    def sub_and_store(j, slot):
      bs, bd, _, _, sw = bufs[slot]

      def sub_row(r, c2):
        bs[r, pl.ds(0, 16)] = bs[r, pl.ds(0, 16)] - bd[r, pl.ds(0, 16)]
        return c2

      lax.fori_loop(0, CHUNK, sub_row, 0)
      pltpu.async_copy(bs, out_slice(j), sw)

    def w_drain(j, slot):
      bs, _, _, _, sw = bufs[slot]
      pltpu.make_async_copy(bs, out_slice(j), sw).wait()

    g_issue(0, 0)

    def pair_body(t, carry):
      j0 = 2 * t
      j1 = j0 + 1
      pl.when(t > 0)(lambda: w_drain(j1 - 2, 1))
      g_issue(j1, 1)
      g_wait(j0, 0)
      sub_and_store(j0, 0)
      w_drain(j0, 0)
      pl.when(t < ROWS_W // 2 - 1)(lambda: g_issue(j0 + 2, 0))
      g_wait(j1, 1)
      sub_and_store(j1, 1)
      return carry

    lax.fori_loop(0, ROWS_W // 2, pair_body, 0)
    w_drain(ROWS_W - 1, 1)

  return k(T, srcs, dsts)


def _sc_segment_sum(e_new, dsts):
  @functools.partial(
      pl.kernel,
      out_type=jax.ShapeDtypeStruct((NC, ACC_ROWS, LATENT), f32),
      mesh=_mesh(),
      scratch_types=[
          pltpu.VMEM((ROWS_W, CHUNK), i32),
          pltpu.VMEM((CHUNK, LATENT), f32),
          pltpu.VMEM((CHUNK, LATENT), f32),
          pltpu.VMEM_SHARED((ACC_ROWS, LATENT), f32),
          pltpu.SemaphoreType.DMA,
          pltpu.SemaphoreType.DMA,
      ],
  )
  def k(e_hbm, dsts_hbm, out_hbm, idx_d, eb0, eb1, acc, sr0, sr1):
    c = lax.axis_index("c")
    s = lax.axis_index("s")
    w = c * NS + s
    pltpu.sync_copy(dsts_hbm.at[pl.ds(w * ROWS_W, ROWS_W)], idx_d)

    bufs = ((eb0, sr0), (eb1, sr1))

    def e_slice(j):
      return e_hbm.at[pl.ds(w * EW + j * CHUNK, CHUNK)]

    def r_issue(j, slot):
      eb, sr = bufs[slot]
      pltpu.async_copy(e_slice(j), eb, sr)

    def r_wait(j, slot):
      eb, sr = bufs[slot]
      pltpu.make_async_copy(e_slice(j), eb, sr).wait()

    def zero_row(r, carry):
      for q in range(LATENT // 16):
        eb0[r, pl.ds(q * 16, 16)] = jnp.zeros((16,), f32)
      return carry

    lax.fori_loop(0, CHUNK, zero_row, 0)
    for q in range(ROWS_T // CHUNK):
      pltpu.sync_copy(eb0, acc.at[pl.ds(s * ROWS_T + q * CHUNK, CHUNK)])
    plsc.subcore_barrier()

    r_issue(0, 0)

    def pair_body(t, carry):
      j0 = 2 * t
      j1 = j0 + 1
      r_issue(j1, 1)
      r_wait(j0, 0)
      pltpu.sync_copy(eb0, acc.at[idx_d.at[j0]], add=True)
      pl.when(t < ROWS_W // 2 - 1)(lambda: r_issue(j0 + 2, 0))
      r_wait(j1, 1)
      pltpu.sync_copy(eb1, acc.at[idx_d.at[j1]], add=True)
      return carry

    lax.fori_loop(0, ROWS_W // 2, pair_body, 0)
    plsc.subcore_barrier()

    for q in range(ROWS_T // CHUNK):
      r = s * ROWS_T + q * CHUNK
      pltpu.sync_copy(acc.at[pl.ds(r, CHUNK)], out_hbm.at[c, pl.ds(r, CHUNK)])

  return k(e_new, dsts)


BE = 2048
BN = 2000


def _ln(h, g, b):
  mu = jnp.mean(h, axis=-1, keepdims=True)
  xc = h - mu
  var = jnp.mean(xc * xc, axis=-1, keepdims=True)
  return xc * lax.rsqrt(var + 1e-5) * g + b


def _dot(a, b):
  return jnp.dot(a, b, preferred_element_type=f32)


def _full(shape):
  return pl.BlockSpec(shape, lambda i: tuple(0 for _ in shape))


def _tc_edge_encoder(diff, W0, W1, W2, consts):
  def body(d_ref, w0_ref, w1_ref, w2_ref, c_ref, out_ref):
    d = d_ref[...]
    rw = d[:, 0:3]
    rm = d[:, 3:6]
    nw = jnp.sqrt(jnp.sum(rw * rw, axis=-1, keepdims=True) + 1e-12)
    nm = jnp.sqrt(jnp.sum(rm * rm, axis=-1, keepdims=True) + 1e-12)
    feat = jnp.concatenate([rw, nw, rm, nm], axis=-1)
    h0 = jnp.maximum(_dot(feat, w0_ref[...]) + c_ref[0], 0.0)
    h1 = jnp.maximum(_dot(h0, w1_ref[...]) + c_ref[1], 0.0)
    h2 = _dot(h1, w2_ref[...]) + c_ref[2]
    out_ref[...] = _ln(h2, c_ref[3], c_ref[4])

  return pl.pallas_call(
      body,
      grid=(E_PAD // BE,),
      in_specs=[
          pl.BlockSpec((BE, 16), lambda i: (i, 0)),
          _full((8, LATENT)),
          _full((LATENT, LATENT)),
          _full((LATENT, LATENT)),
          _full((8, LATENT)),
      ],
      out_specs=pl.BlockSpec((BE, LATENT), lambda i: (i, 0)),
      out_shape=jax.ShapeDtypeStruct((E_PAD, LATENT), f32),
  )(diff, W0, W1, W2, consts)


def _tc_node_encoder(wp, pwp, tcol, W0, W1, W2, consts, Wp):
  def body(wp_ref, pwp_ref, t_ref, w0_ref, w1_ref, w2_ref, c_ref, wp_proj_ref,
           nlat_ref, ps_ref, pd_ref):
    vel = wp_ref[...] - pwp_ref[...]
    t = t_ref[...].astype(i32)
    oh = jnp.where(
        t == lax.broadcasted_iota(i32, (BN, NUM_TYPES), 1), 1.0, 0.0)
    feat = jnp.concatenate([vel, oh], axis=-1)
    h0 = jnp.maximum(_dot(feat, w0_ref[...]) + c_ref[0], 0.0)
    h1 = jnp.maximum(_dot(h0, w1_ref[...]) + c_ref[1], 0.0)
    h2 = _dot(h1, w2_ref[...]) + c_ref[2]
    nl = _ln(h2, c_ref[3], c_ref[4])
    nlat_ref[...] = nl
    proj = _dot(nl, wp_proj_ref[...])
    ps_ref[...] = proj[:, :LATENT]
    pd_ref[...] = proj[:, LATENT:]

  return pl.pallas_call(
      body,
      grid=(N // BN,),
      in_specs=[
          pl.BlockSpec((BN, 3), lambda i: (i, 0)),
          pl.BlockSpec((BN, 3), lambda i: (i, 0)),
          pl.BlockSpec((BN, 1), lambda i: (i, 0)),
          _full((NUM_TYPES + 3, LATENT)),
          _full((LATENT, LATENT)),
          _full((LATENT, LATENT)),
          _full((8, LATENT)),
          _full((LATENT, 2 * LATENT)),
      ],
      out_specs=[
          pl.BlockSpec((BN, LATENT), lambda i: (i, 0)),
          pl.BlockSpec((BN, LATENT), lambda i: (i, 0)),
          pl.BlockSpec((BN, LATENT), lambda i: (i, 0)),
      ],
      out_shape=[jax.ShapeDtypeStruct((N, LATENT), f32)] * 3,
  )(wp, pwp, tcol, W0, W1, W2, consts, Wp)


def _tc_edge_block(gsum, elat, W0e, W1, W2, consts):
  def body(g_ref, e_ref, w0_ref, w1_ref, w2_ref, c_ref, en_ref, eo_ref):
    e = e_ref[...]
    h0 = jnp.maximum(g_ref[...] + _dot(e, w0_ref[...]) + c_ref[0], 0.0)
    h1 = jnp.maximum(_dot(h0, w1_ref[...]) + c_ref[1], 0.0)
    h2 = _dot(h1, w2_ref[...]) + c_ref[2]
    y = _ln(h2, c_ref[3], c_ref[4])
    en_ref[...] = y
    eo_ref[...] = e + y

  return pl.pallas_call(
      body,
      grid=(E_PAD // BE,),
      in_specs=[
          pl.BlockSpec((BE, LATENT), lambda i: (i, 0)),
          pl.BlockSpec((BE, LATENT), lambda i: (i, 0)),
          _full((LATENT, LATENT)),
          _full((LATENT, LATENT)),
          _full((LATENT, LATENT)),
          _full((8, LATENT)),
      ],
      out_specs=[
          pl.BlockSpec((BE, LATENT), lambda i: (i, 0)),
          pl.BlockSpec((BE, LATENT), lambda i: (i, 0)),
      ],
      out_shape=[jax.ShapeDtypeStruct((E_PAD, LATENT), f32)] * 2,
  )(gsum, elat, W0e, W1, W2, consts)


def _tc_node_block(nlat, agg2, W0, W1, W2, consts, Wp):
  def body(n_ref, a_ref, w0_ref, w1_ref, w2_ref, c_ref, wp_ref,
           no_ref, ps_ref, pd_ref):
    nl = n_ref[...]
    agg = a_ref[0] + a_ref[1]
    x = jnp.concatenate([nl, agg], axis=-1)
    h0 = jnp.maximum(_dot(x, w0_ref[...]) + c_ref[0], 0.0)
    h1 = jnp.maximum(_dot(h0, w1_ref[...]) + c_ref[1], 0.0)
    h2 = _dot(h1, w2_ref[...]) + c_ref[2]
    nl_new = nl + _ln(h2, c_ref[3], c_ref[4])
    no_ref[...] = nl_new
    proj = _dot(nl_new, wp_ref[...])
    ps_ref[...] = proj[:, :LATENT]
    pd_ref[...] = proj[:, LATENT:]

  return pl.pallas_call(
      body,
      grid=(N // BN,),
      in_specs=[
          pl.BlockSpec((BN, LATENT), lambda i: (i, 0)),
          pl.BlockSpec((NC, BN, LATENT), lambda i: (0, i, 0)),
          _full((2 * LATENT, LATENT)),
          _full((LATENT, LATENT)),
          _full((LATENT, LATENT)),
          _full((8, LATENT)),
          _full((LATENT, 2 * LATENT)),
      ],
      out_specs=[
          pl.BlockSpec((BN, LATENT), lambda i: (i, 0)),
          pl.BlockSpec((BN, LATENT), lambda i: (i, 0)),
          pl.BlockSpec((BN, LATENT), lambda i: (i, 0)),
      ],
      out_shape=[jax.ShapeDtypeStruct((N, LATENT), f32)] * 3,
  )(nlat, agg2, W0, W1, W2, consts, Wp)


def _tc_node_block_last(nlat, agg2, W0, W1, W2, consts):
  def body(n_ref, a_ref, w0_ref, w1_ref, w2_ref, c_ref, no_ref):
    nl = n_ref[...]
    agg = a_ref[0] + a_ref[1]
    x = jnp.concatenate([nl, agg], axis=-1)
    h0 = jnp.maximum(_dot(x, w0_ref[...]) + c_ref[0], 0.0)
    h1 = jnp.maximum(_dot(h0, w1_ref[...]) + c_ref[1], 0.0)
    h2 = _dot(h1, w2_ref[...]) + c_ref[2]
    no_ref[...] = nl + _ln(h2, c_ref[3], c_ref[4])

  return pl.pallas_call(
      body,
      grid=(N // BN,),
      in_specs=[
          pl.BlockSpec((BN, LATENT), lambda i: (i, 0)),
          pl.BlockSpec((NC, BN, LATENT), lambda i: (0, i, 0)),
          _full((2 * LATENT, LATENT)),
          _full((LATENT, LATENT)),
          _full((LATENT, LATENT)),
          _full((8, LATENT)),
      ],
      out_specs=pl.BlockSpec((BN, LATENT), lambda i: (i, 0)),
      out_shape=jax.ShapeDtypeStruct((N, LATENT), f32),
  )(nlat, agg2, W0, W1, W2, consts)


def _tc_decoder(nlat, wp, pwp, tcol, W0, W1, W2p, consts):
  def body(n_ref, wp_ref, pwp_ref, t_ref, w0_ref, w1_ref, w2_ref, c_ref,
           out_ref):
    h0 = jnp.maximum(_dot(n_ref[...], w0_ref[...]) + c_ref[0], 0.0)
    h1 = jnp.maximum(_dot(h0, w1_ref[...]) + c_ref[1], 0.0)
    h2 = _dot(h1, w2_ref[...]) + c_ref[2]
    acc = h2 * c_ref[3] + c_ref[4]
    wpv = wp_ref[...]
    pred_pos = 2.0 * wpv + acc[:, 0:3] - pwp_ref[...]
    mask = t_ref[...] == 0.0
    out_ref[...] = jnp.where(mask, pred_pos, wpv)

  return pl.pallas_call(
      body,
      grid=(N // BN,),
      in_specs=[
          pl.BlockSpec((BN, LATENT), lambda i: (i, 0)),
          pl.BlockSpec((BN, 3), lambda i: (i, 0)),
          pl.BlockSpec((BN, 3), lambda i: (i, 0)),
          pl.BlockSpec((BN, 1), lambda i: (i, 0)),
          _full((LATENT, LATENT)),
          _full((LATENT, LATENT)),
          _full((LATENT, LATENT)),
          _full((8, LATENT)),
      ],
      out_specs=pl.BlockSpec((BN, 3), lambda i: (i, 0)),
      out_shape=jax.ShapeDtypeStruct((N, 3), f32),
  )(nlat, wp, pwp, tcol, W0, W1, W2p, consts)


def _pack_consts(b0, b1, b2, g=None, b=None):
  rows = [b0, b1, b2]
  rows.append(g if g is not None else jnp.zeros((LATENT,), f32))
  rows.append(b if b is not None else jnp.zeros((LATENT,), f32))
  rows += [jnp.zeros((LATENT,), f32)] * 3
  return jnp.stack([jnp.pad(r, (0, LATENT - r.shape[0])) for r in rows])


def kernel(world_pos, prev_world_pos, mesh_pos, params, node_type, edge_index):
  src = edge_index[0].astype(i32)
  dst = edge_index[1].astype(i32)
  pad = E_PAD - E
  src_g = jnp.concatenate([src, jnp.zeros((pad,), i32)]).reshape(-1, CHUNK)
  dst_g = jnp.concatenate([dst, jnp.zeros((pad,), i32)]).reshape(-1, CHUNK)
  dst_s = jnp.concatenate([dst, jnp.full((pad,), N, i32)]).reshape(-1, CHUNK)

  T = jnp.concatenate(
      [world_pos, mesh_pos, jnp.zeros((N, 10), f32)], axis=1)
  tcol = node_type.astype(f32)[:, None]

  p = params

  def fold_first(mlp, mean, std):
    w0 = mlp['W0'] / std[:, None]
    b0 = mlp['b0'] - jnp.dot(mean / std, mlp['W0'])
    return w0, b0

  ew0, eb0 = fold_first(p['enc_edge'], p['mesh_norm']['mean'],
                        p['mesh_norm']['std'])
  nw0, nb0 = fold_first(p['enc_node'], p['node_norm']['mean'],
                        p['node_norm']['std'])
  enc_e = p['enc_edge']
  enc_n = p['enc_node']

  diff = _sc_feature_diff(T, src_g, dst_g)
  elat = _tc_edge_encoder(
      diff, ew0, enc_e['W1'], enc_e['W2'],
      _pack_consts(eb0, enc_e['b1'], enc_e['b2'], enc_e['ln_g'],
                   enc_e['ln_b']))

  def proj_weights(blk):
    w0 = blk['edge_mlp']['W0']
    return jnp.concatenate([w0[:LATENT], w0[LATENT:2 * LATENT]], axis=1)

  nlat, Ps, Pd = _tc_node_encoder(
      world_pos, prev_world_pos, tcol, nw0, enc_n['W1'], enc_n['W2'],
      _pack_consts(nb0, enc_n['b1'], enc_n['b2'], enc_n['ln_g'],
                   enc_n['ln_b']),
      proj_weights(p['blocks'][0]))

  for b in range(len(p['blocks'])):
    blk = p['blocks'][b]
    em = blk['edge_mlp']
    nm = blk['node_mlp']
    gsum = _sc_gather_sum(Ps, Pd, src_g, dst_g)
    e_new, elat = _tc_edge_block(
        gsum, elat, em['W0'][2 * LATENT:], em['W1'], em['W2'],
        _pack_consts(em['b0'], em['b1'], em['b2'], em['ln_g'], em['ln_b']))
    agg2 = _sc_segment_sum(e_new, dst_s)
    nconsts = _pack_consts(nm['b0'], nm['b1'], nm['b2'], nm['ln_g'],
                           nm['ln_b'])
    if b + 1 < len(p['blocks']):
      nlat, Ps, Pd = _tc_node_block(
          nlat, agg2, nm['W0'], nm['W1'], nm['W2'], nconsts,
          proj_weights(p['blocks'][b + 1]))
    else:
      nlat = _tc_node_block_last(
          nlat, agg2, nm['W0'], nm['W1'], nm['W2'], nconsts)

  dec = p['dec']
  W2p = jnp.pad(dec['W2'], ((0, 0), (0, LATENT - 3)))
  dconsts = _pack_consts(
      dec['b0'], dec['b1'], dec['b2'],
      jnp.pad(p['out_norm']['std'], (0, LATENT - 3), constant_values=1.0),
      jnp.pad(p['out_norm']['mean'], (0, LATENT - 3)))
  return _tc_decoder(nlat, world_pos, prev_world_pos, tcol,
                     dec['W0'], dec['W1'], W2p, dconsts)


# speedup vs baseline: 1.4875x; 1.1076x over previous
import functools

import jax
import jax.numpy as jnp
from jax import lax
from jax.experimental import pallas as pl
from jax.experimental.pallas import tpu as pltpu
from jax.experimental.pallas import tpu_sc as plsc

N = 10000
E = 160000
LATENT = 128
NUM_TYPES = 9

NC = 2
NS = 16
NW = NC * NS
CHUNK = 128
E_PAD = 163840
ROWS_W = E_PAD // NW // CHUNK
EW = E_PAD // NW
ACC_ROWS = 10240
ROWS_T = ACC_ROWS // NS

@functools.cache
def _mesh():
  return plsc.VectorSubcoreMesh(
      core_axis_name="c", subcore_axis_name="s", num_cores=NC,
      num_subcores=NS)

f32 = jnp.float32
i32 = jnp.int32
bf16 = jnp.bfloat16

_PI = tuple(list(range(0, LATENT, 2)) + list(range(1, LATENT, 2)))


_NSLOT = 4


def _sc_gather_sum(Ps, Pd, srcs, dsts):
  Ps = jax.lax.bitcast_convert_type(Ps.reshape(N, LATENT // 2, 2), i32)
  Pd = jax.lax.bitcast_convert_type(Pd.reshape(N, LATENT // 2, 2), i32)

  @functools.partial(
      pl.kernel,
      out_type=jax.ShapeDtypeStruct((E_PAD, LATENT), i32),
      mesh=_mesh(),
      scratch_types=[
          pltpu.VMEM((ROWS_W, CHUNK), i32),
          pltpu.VMEM((ROWS_W, CHUNK), i32),
      ] + [pltpu.VMEM((CHUNK, LATENT // 2), i32)] * (2 * _NSLOT)
        + [pltpu.SemaphoreType.DMA] * (3 * _NSLOT),
      compiler_params=pltpu.CompilerParams(use_tc_tiling_on_sc=False),
  )
  def k(ps_hbm, pd_hbm, srcs_hbm, dsts_hbm, out_hbm, idx_s, idx_d, *rest):
    bufs_s = rest[0:_NSLOT]
    bufs_d = rest[_NSLOT:2 * _NSLOT]
    sems_s = rest[2 * _NSLOT:3 * _NSLOT]
    sems_d = rest[3 * _NSLOT:4 * _NSLOT]
    sems_w = rest[4 * _NSLOT:5 * _NSLOT]
    w = lax.axis_index("c") * NS + lax.axis_index("s")
    r0 = w * ROWS_W
    pltpu.sync_copy(srcs_hbm.at[pl.ds(r0, ROWS_W)], idx_s)
    pltpu.sync_copy(dsts_hbm.at[pl.ds(r0, ROWS_W)], idx_d)

    def g_issue(j, b):
      pltpu.async_copy(ps_hbm.at[idx_s.at[j]], bufs_s[b], sems_s[b])
      pltpu.async_copy(pd_hbm.at[idx_d.at[j]], bufs_d[b], sems_d[b])

    def g_wait(j, b):
      pltpu.make_async_copy(ps_hbm.at[idx_s.at[j]], bufs_s[b],
                            sems_s[b]).wait()
      pltpu.make_async_copy(pd_hbm.at[idx_d.at[j]], bufs_d[b],
                            sems_d[b]).wait()

    def out_s(j):
      return out_hbm.at[pl.ds(w * EW + j * CHUNK, CHUNK),
                        pl.ds(0, LATENT // 2)]

    def out_d(j):
      return out_hbm.at[pl.ds(w * EW + j * CHUNK, CHUNK),
                        pl.ds(LATENT // 2, LATENT // 2)]

    def w_issue(j, b):
      pltpu.async_copy(bufs_s[b], out_s(j), sems_w[b])
      pltpu.async_copy(bufs_d[b], out_d(j), sems_w[b])

    def w_drain(j, b):
      pltpu.make_async_copy(bufs_s[b], out_s(j), sems_w[b]).wait()
      pltpu.make_async_copy(bufs_d[b], out_d(j), sems_w[b]).wait()

    for b in range(_NSLOT):
      g_issue(b, b)
    n_grp = ROWS_W // _NSLOT

    def group_body(t, carry):
      for b in range(_NSLOT):
        j = _NSLOT * t + b
        g_wait(j, b)
        w_issue(j, b)
        w_drain(j, b)
        pl.when(t < n_grp - 1)(functools.partial(g_issue, j + _NSLOT, b))
      return carry

    lax.fori_loop(0, n_grp, group_body, 0)

  return k(Ps, Pd, srcs, dsts)


def _sc_feature_diff(T, srcs, dsts):
  D = 16

  @functools.partial(
      pl.kernel,
      out_type=jax.ShapeDtypeStruct((E_PAD, D), f32),
      mesh=_mesh(),
      scratch_types=[
          pltpu.VMEM((ROWS_W, CHUNK), i32),
          pltpu.VMEM((ROWS_W, CHUNK), i32),
          pltpu.VMEM((CHUNK, D), f32),
          pltpu.VMEM((CHUNK, D), f32),
          pltpu.VMEM((CHUNK, D), f32),
          pltpu.VMEM((CHUNK, D), f32),
          pltpu.SemaphoreType.DMA,
          pltpu.SemaphoreType.DMA,
          pltpu.SemaphoreType.DMA,
          pltpu.SemaphoreType.DMA,
          pltpu.SemaphoreType.DMA,
          pltpu.SemaphoreType.DMA,
      ],
      compiler_params=pltpu.CompilerParams(use_tc_tiling_on_sc=False),
  )
  def k(t_hbm, srcs_hbm, dsts_hbm, out_hbm, idx_s, idx_d,
        bs0, bd0, bs1, bd1, ss0, sd0, ss1, sd1, sw0, sw1):
    w = lax.axis_index("c") * NS + lax.axis_index("s")
    r0 = w * ROWS_W
    pltpu.sync_copy(srcs_hbm.at[pl.ds(r0, ROWS_W)], idx_s)
    pltpu.sync_copy(dsts_hbm.at[pl.ds(r0, ROWS_W)], idx_d)

    bufs = ((bs0, bd0, ss0, sd0, sw0), (bs1, bd1, ss1, sd1, sw1))

    def g_issue(j, slot):
      bs, bd, ss, sd, _ = bufs[slot]
      pltpu.async_copy(t_hbm.at[idx_s.at[j]], bs, ss)
      pltpu.async_copy(t_hbm.at[idx_d.at[j]], bd, sd)

    def g_wait(j, slot):
      bs, bd, ss, sd, _ = bufs[slot]
      pltpu.make_async_copy(t_hbm.at[idx_s.at[j]], bs, ss).wait()
      pltpu.make_async_copy(t_hbm.at[idx_d.at[j]], bd, sd).wait()

    def out_slice(j):
      return out_hbm.at[pl.ds(w * EW + j * CHUNK, CHUNK)]

    def sub_and_store(j, slot):
      bs, bd, _, _, sw = bufs[slot]

      def sub_row(r, c2):
        bs[r, pl.ds(0, 16)] = bs[r, pl.ds(0, 16)] - bd[r, pl.ds(0, 16)]
        return c2

      lax.fori_loop(0, CHUNK, sub_row, 0)
      pltpu.async_copy(bs, out_slice(j), sw)

    def w_drain(j, slot):
      bs, _, _, _, sw = bufs[slot]
      pltpu.make_async_copy(bs, out_slice(j), sw).wait()

    g_issue(0, 0)

    def pair_body(t, carry):
      j0 = 2 * t
      j1 = j0 + 1
      pl.when(t > 0)(lambda: w_drain(j1 - 2, 1))
      g_issue(j1, 1)
      g_wait(j0, 0)
      sub_and_store(j0, 0)
      w_drain(j0, 0)
      pl.when(t < ROWS_W // 2 - 1)(lambda: g_issue(j0 + 2, 0))
      g_wait(j1, 1)
      sub_and_store(j1, 1)
      return carry

    lax.fori_loop(0, ROWS_W // 2, pair_body, 0)
    w_drain(ROWS_W - 1, 1)

  return k(T, srcs, dsts)


def _sc_segment_sum(e_new, dsts):
  @functools.partial(
      pl.kernel,
      out_type=jax.ShapeDtypeStruct((NC, ACC_ROWS, LATENT), f32),
      mesh=_mesh(),
      scratch_types=[
          pltpu.VMEM((ROWS_W, CHUNK), i32),
          pltpu.VMEM((CHUNK, LATENT), f32),
          pltpu.VMEM((CHUNK, LATENT), f32),
          pltpu.VMEM_SHARED((ACC_ROWS, LATENT), f32),
          pltpu.SemaphoreType.DMA,
          pltpu.SemaphoreType.DMA,
      ],
  )
  def k(e_hbm, dsts_hbm, out_hbm, idx_d, eb0, eb1, acc, sr0, sr1):
    c = lax.axis_index("c")
    s = lax.axis_index("s")
    w = c * NS + s
    pltpu.sync_copy(dsts_hbm.at[pl.ds(w * ROWS_W, ROWS_W)], idx_d)

    bufs = ((eb0, sr0), (eb1, sr1))

    def e_slice(j):
      return e_hbm.at[pl.ds(w * EW + j * CHUNK, CHUNK)]

    def r_issue(j, slot):
      eb, sr = bufs[slot]
      pltpu.async_copy(e_slice(j), eb, sr)

    def r_wait(j, slot):
      eb, sr = bufs[slot]
      pltpu.make_async_copy(e_slice(j), eb, sr).wait()

    def zero_row(r, carry):
      for q in range(LATENT // 16):
        eb0[r, pl.ds(q * 16, 16)] = jnp.zeros((16,), f32)
      return carry

    lax.fori_loop(0, CHUNK, zero_row, 0)
    for q in range(ROWS_T // CHUNK):
      pltpu.sync_copy(eb0, acc.at[pl.ds(s * ROWS_T + q * CHUNK, CHUNK)])
    plsc.subcore_barrier()

    r_issue(0, 0)

    def pair_body(t, carry):
      j0 = 2 * t
      j1 = j0 + 1
      r_issue(j1, 1)
      r_wait(j0, 0)
      pltpu.sync_copy(eb0, acc.at[idx_d.at[j0]], add=True)
      pl.when(t < ROWS_W // 2 - 1)(lambda: r_issue(j0 + 2, 0))
      r_wait(j1, 1)
      pltpu.sync_copy(eb1, acc.at[idx_d.at[j1]], add=True)
      return carry

    lax.fori_loop(0, ROWS_W // 2, pair_body, 0)
    plsc.subcore_barrier()

    for q in range(ROWS_T // CHUNK):
      r = s * ROWS_T + q * CHUNK
      pltpu.sync_copy(acc.at[pl.ds(r, CHUNK)], out_hbm.at[c, pl.ds(r, CHUNK)])

  return k(e_new, dsts)


BE = 2048
BN = 2000


def _ln(h, g, b):
  mu = jnp.mean(h, axis=-1, keepdims=True)
  xc = h - mu
  var = jnp.mean(xc * xc, axis=-1, keepdims=True)
  return xc * lax.rsqrt(var + 1e-5) * g + b


def _dot(a, b):
  return jnp.dot(a, b, preferred_element_type=f32)


def _full(shape):
  return pl.BlockSpec(shape, lambda i: tuple(0 for _ in shape))


def _tc_edge_encoder(diff, W0, W1, W2, consts):
  def body(d_ref, w0_ref, w1_ref, w2_ref, c_ref, out_ref):
    d = d_ref[...]
    rw = d[:, 0:3]
    rm = d[:, 3:6]
    nw = jnp.sqrt(jnp.sum(rw * rw, axis=-1, keepdims=True) + 1e-12)
    nm = jnp.sqrt(jnp.sum(rm * rm, axis=-1, keepdims=True) + 1e-12)
    feat = jnp.concatenate([rw, nw, rm, nm], axis=-1)
    h0 = jnp.maximum(_dot(feat, w0_ref[...]) + c_ref[0], 0.0)
    h1 = jnp.maximum(_dot(h0, w1_ref[...]) + c_ref[1], 0.0)
    h2 = _dot(h1, w2_ref[...]) + c_ref[2]
    out_ref[...] = _ln(h2, c_ref[3], c_ref[4])

  return pl.pallas_call(
      body,
      grid=(E_PAD // BE,),
      in_specs=[
          pl.BlockSpec((BE, 16), lambda i: (i, 0)),
          _full((8, LATENT)),
          _full((LATENT, LATENT)),
          _full((LATENT, LATENT)),
          _full((8, LATENT)),
      ],
      out_specs=pl.BlockSpec((BE, LATENT), lambda i: (i, 0)),
      out_shape=jax.ShapeDtypeStruct((E_PAD, LATENT), f32),
  )(diff, W0, W1, W2, consts)


def _tc_node_encoder(wp, pwp, tcol, W0, W1, W2, consts, Wp):
  def body(wp_ref, pwp_ref, t_ref, w0_ref, w1_ref, w2_ref, c_ref, wp_proj_ref,
           nlat_ref, ps_ref, pd_ref):
    vel = wp_ref[...] - pwp_ref[...]
    t = t_ref[...].astype(i32)
    oh = jnp.where(
        t == lax.broadcasted_iota(i32, (BN, NUM_TYPES), 1), 1.0, 0.0)
    feat = jnp.concatenate([vel, oh], axis=-1)
    h0 = jnp.maximum(_dot(feat, w0_ref[...]) + c_ref[0], 0.0)
    h1 = jnp.maximum(_dot(h0, w1_ref[...]) + c_ref[1], 0.0)
    h2 = _dot(h1, w2_ref[...]) + c_ref[2]
    nl = _ln(h2, c_ref[3], c_ref[4])
    nlat_ref[...] = nl
    proj = _dot(nl, wp_proj_ref[...])
    ps_ref[...] = proj[:, :LATENT].astype(bf16)
    pd_ref[...] = proj[:, LATENT:].astype(bf16)

  return pl.pallas_call(
      body,
      grid=(N // BN,),
      in_specs=[
          pl.BlockSpec((BN, 3), lambda i: (i, 0)),
          pl.BlockSpec((BN, 3), lambda i: (i, 0)),
          pl.BlockSpec((BN, 1), lambda i: (i, 0)),
          _full((NUM_TYPES + 3, LATENT)),
          _full((LATENT, LATENT)),
          _full((LATENT, LATENT)),
          _full((8, LATENT)),
          _full((LATENT, 2 * LATENT)),
      ],
      out_specs=[
          pl.BlockSpec((BN, LATENT), lambda i: (i, 0)),
          pl.BlockSpec((BN, LATENT), lambda i: (i, 0)),
          pl.BlockSpec((BN, LATENT), lambda i: (i, 0)),
      ],
      out_shape=[jax.ShapeDtypeStruct((N, LATENT), f32),
                 jax.ShapeDtypeStruct((N, LATENT), bf16),
                 jax.ShapeDtypeStruct((N, LATENT), bf16)],
  )(wp, pwp, tcol, W0, W1, W2, consts, Wp)


def _tc_edge_block(gsum, elat, W0e, W1, W2, consts):
  def body(g_ref, e_ref, w0_ref, w1_ref, w2_ref, c_ref, en_ref, eo_ref):
    e = e_ref[...]
    gi = g_ref[...]
    xs = gi[:, :LATENT // 2]
    xd = gi[:, LATENT // 2:]

    def lo(x):
      return jax.lax.bitcast_convert_type(x << 16, f32)

    def hi(x):
      return jax.lax.bitcast_convert_type(x & jnp.int32(-65536), f32)

    gp = jnp.concatenate([lo(xs) + lo(xd), hi(xs) + hi(xd)], axis=1)
    h0 = jnp.maximum(gp + _dot(e, w0_ref[...]) + c_ref[0], 0.0)
    h1 = jnp.maximum(_dot(h0, w1_ref[...]) + c_ref[1], 0.0)
    h2 = _dot(h1, w2_ref[...]) + c_ref[2]
    y = _ln(h2, c_ref[3], c_ref[4])
    en_ref[...] = y
    eo_ref[...] = e + y

  return pl.pallas_call(
      body,
      grid=(E_PAD // BE,),
      in_specs=[
          pl.BlockSpec((BE, LATENT), lambda i: (i, 0)),
          pl.BlockSpec((BE, LATENT), lambda i: (i, 0)),
          _full((LATENT, LATENT)),
          _full((LATENT, LATENT)),
          _full((LATENT, LATENT)),
          _full((8, LATENT)),
      ],
      out_specs=[
          pl.BlockSpec((BE, LATENT), lambda i: (i, 0)),
          pl.BlockSpec((BE, LATENT), lambda i: (i, 0)),
      ],
      out_shape=[jax.ShapeDtypeStruct((E_PAD, LATENT), f32)] * 2,
  )(gsum, elat, W0e, W1, W2, consts)


def _tc_node_block(nlat, agg2, W0, W1, W2, consts, Wp):
  def body(n_ref, a_ref, w0_ref, w1_ref, w2_ref, c_ref, wp_ref,
           no_ref, ps_ref, pd_ref):
    nl = n_ref[...]
    agg = a_ref[0] + a_ref[1]
    x = jnp.concatenate([nl, agg], axis=-1)
    h0 = jnp.maximum(_dot(x, w0_ref[...]) + c_ref[0], 0.0)
    h1 = jnp.maximum(_dot(h0, w1_ref[...]) + c_ref[1], 0.0)
    h2 = _dot(h1, w2_ref[...]) + c_ref[2]
    nl_new = nl + _ln(h2, c_ref[3], c_ref[4])
    no_ref[...] = nl_new
    proj = _dot(nl_new, wp_ref[...])
    ps_ref[...] = proj[:, :LATENT].astype(bf16)
    pd_ref[...] = proj[:, LATENT:].astype(bf16)

  return pl.pallas_call(
      body,
      grid=(N // BN,),
      in_specs=[
          pl.BlockSpec((BN, LATENT), lambda i: (i, 0)),
          pl.BlockSpec((NC, BN, LATENT), lambda i: (0, i, 0)),
          _full((2 * LATENT, LATENT)),
          _full((LATENT, LATENT)),
          _full((LATENT, LATENT)),
          _full((8, LATENT)),
          _full((LATENT, 2 * LATENT)),
      ],
      out_specs=[
          pl.BlockSpec((BN, LATENT), lambda i: (i, 0)),
          pl.BlockSpec((BN, LATENT), lambda i: (i, 0)),
          pl.BlockSpec((BN, LATENT), lambda i: (i, 0)),
      ],
      out_shape=[jax.ShapeDtypeStruct((N, LATENT), f32),
                 jax.ShapeDtypeStruct((N, LATENT), bf16),
                 jax.ShapeDtypeStruct((N, LATENT), bf16)],
  )(nlat, agg2, W0, W1, W2, consts, Wp)


def _tc_node_block_last(nlat, agg2, W0, W1, W2, consts):
  def body(n_ref, a_ref, w0_ref, w1_ref, w2_ref, c_ref, no_ref):
    nl = n_ref[...]
    agg = a_ref[0] + a_ref[1]
    x = jnp.concatenate([nl, agg], axis=-1)
    h0 = jnp.maximum(_dot(x, w0_ref[...]) + c_ref[0], 0.0)
    h1 = jnp.maximum(_dot(h0, w1_ref[...]) + c_ref[1], 0.0)
    h2 = _dot(h1, w2_ref[...]) + c_ref[2]
    no_ref[...] = nl + _ln(h2, c_ref[3], c_ref[4])

  return pl.pallas_call(
      body,
      grid=(N // BN,),
      in_specs=[
          pl.BlockSpec((BN, LATENT), lambda i: (i, 0)),
          pl.BlockSpec((NC, BN, LATENT), lambda i: (0, i, 0)),
          _full((2 * LATENT, LATENT)),
          _full((LATENT, LATENT)),
          _full((LATENT, LATENT)),
          _full((8, LATENT)),
      ],
      out_specs=pl.BlockSpec((BN, LATENT), lambda i: (i, 0)),
      out_shape=jax.ShapeDtypeStruct((N, LATENT), f32),
  )(nlat, agg2, W0, W1, W2, consts)


def _tc_decoder(nlat, wp, pwp, tcol, W0, W1, W2p, consts):
  def body(n_ref, wp_ref, pwp_ref, t_ref, w0_ref, w1_ref, w2_ref, c_ref,
           out_ref):
    h0 = jnp.maximum(_dot(n_ref[...], w0_ref[...]) + c_ref[0], 0.0)
    h1 = jnp.maximum(_dot(h0, w1_ref[...]) + c_ref[1], 0.0)
    h2 = _dot(h1, w2_ref[...]) + c_ref[2]
    acc = h2 * c_ref[3] + c_ref[4]
    wpv = wp_ref[...]
    pred_pos = 2.0 * wpv + acc[:, 0:3] - pwp_ref[...]
    mask = t_ref[...] == 0.0
    out_ref[...] = jnp.where(mask, pred_pos, wpv)

  return pl.pallas_call(
      body,
      grid=(N // BN,),
      in_specs=[
          pl.BlockSpec((BN, LATENT), lambda i: (i, 0)),
          pl.BlockSpec((BN, 3), lambda i: (i, 0)),
          pl.BlockSpec((BN, 3), lambda i: (i, 0)),
          pl.BlockSpec((BN, 1), lambda i: (i, 0)),
          _full((LATENT, LATENT)),
          _full((LATENT, LATENT)),
          _full((LATENT, LATENT)),
          _full((8, LATENT)),
      ],
      out_specs=pl.BlockSpec((BN, 3), lambda i: (i, 0)),
      out_shape=jax.ShapeDtypeStruct((N, 3), f32),
  )(nlat, wp, pwp, tcol, W0, W1, W2p, consts)


def _pack_consts(b0, b1, b2, g=None, b=None):
  rows = [b0, b1, b2]
  rows.append(g if g is not None else jnp.zeros((LATENT,), f32))
  rows.append(b if b is not None else jnp.zeros((LATENT,), f32))
  rows += [jnp.zeros((LATENT,), f32)] * 3
  return jnp.stack([jnp.pad(r, (0, LATENT - r.shape[0])) for r in rows])


def kernel(world_pos, prev_world_pos, mesh_pos, params, node_type, edge_index):
  src = edge_index[0].astype(i32)
  dst = edge_index[1].astype(i32)
  pad = E_PAD - E
  src_g = jnp.concatenate([src, jnp.zeros((pad,), i32)]).reshape(-1, CHUNK)
  dst_g = jnp.concatenate([dst, jnp.zeros((pad,), i32)]).reshape(-1, CHUNK)
  dst_s = jnp.concatenate([dst, jnp.full((pad,), N, i32)]).reshape(-1, CHUNK)

  T = jnp.concatenate(
      [world_pos, mesh_pos, jnp.zeros((N, 10), f32)], axis=1)
  tcol = node_type.astype(f32)[:, None]

  p = params

  def fold_first(mlp, mean, std):
    w0 = mlp['W0'] / std[:, None]
    b0 = mlp['b0'] - jnp.dot(mean / std, mlp['W0'])
    return w0, b0

  ew0, eb0 = fold_first(p['enc_edge'], p['mesh_norm']['mean'],
                        p['mesh_norm']['std'])
  nw0, nb0 = fold_first(p['enc_node'], p['node_norm']['mean'],
                        p['node_norm']['std'])
  enc_e = p['enc_edge']
  enc_n = p['enc_node']

  diff = _sc_feature_diff(T, src_g, dst_g)
  elat = _tc_edge_encoder(
      diff, ew0, enc_e['W1'], enc_e['W2'],
      _pack_consts(eb0, enc_e['b1'], enc_e['b2'], enc_e['ln_g'],
                   enc_e['ln_b']))

  pi = jnp.array(_PI, i32)

  def proj_weights(blk):
    w0 = blk['edge_mlp']['W0']
    return jnp.concatenate([w0[:LATENT], w0[LATENT:2 * LATENT]], axis=1)

  nlat, Ps, Pd = _tc_node_encoder(
      world_pos, prev_world_pos, tcol, nw0, enc_n['W1'], enc_n['W2'],
      _pack_consts(nb0, enc_n['b1'], enc_n['b2'], enc_n['ln_g'],
                   enc_n['ln_b']),
      proj_weights(p['blocks'][0]))

  for b in range(len(p['blocks'])):
    blk = p['blocks'][b]
    em = blk['edge_mlp']
    nm = blk['node_mlp']
    gsum = _sc_gather_sum(Ps, Pd, src_g, dst_g)
    e_new, elat = _tc_edge_block(
        gsum, elat, em['W0'][2 * LATENT:][:, pi], em['W1'][pi, :], em['W2'],
        _pack_consts(em['b0'][pi], em['b1'], em['b2'], em['ln_g'],
                     em['ln_b']))
    agg2 = _sc_segment_sum(e_new, dst_s)
    nconsts = _pack_consts(nm['b0'], nm['b1'], nm['b2'], nm['ln_g'],
                           nm['ln_b'])
    if b + 1 < len(p['blocks']):
      nlat, Ps, Pd = _tc_node_block(
          nlat, agg2, nm['W0'], nm['W1'], nm['W2'], nconsts,
          proj_weights(p['blocks'][b + 1]))
    else:
      nlat = _tc_node_block_last(
          nlat, agg2, nm['W0'], nm['W1'], nm['W2'], nconsts)

  dec = p['dec']
  W2p = jnp.pad(dec['W2'], ((0, 0), (0, LATENT - 3)))
  dconsts = _pack_consts(
      dec['b0'], dec['b1'], dec['b2'],
      jnp.pad(p['out_norm']['std'], (0, LATENT - 3), constant_values=1.0),
      jnp.pad(p['out_norm']['mean'], (0, LATENT - 3)))
  return _tc_decoder(nlat, world_pos, prev_world_pos, tcol,
                     dec['W0'], dec['W1'], W2p, dconsts)


# speedup vs baseline: 1.5273x; 1.0268x over previous
import functools

import jax
import jax.numpy as jnp
from jax import lax
from jax.experimental import pallas as pl
from jax.experimental.pallas import tpu as pltpu
from jax.experimental.pallas import tpu_sc as plsc

N = 10000
E = 160000
LATENT = 128
NUM_TYPES = 9

NC = 2
NS = 16
NW = NC * NS
CHUNK = 128
E_PAD = 163840
ROWS_W = E_PAD // NW // CHUNK
EW = E_PAD // NW
ACC_ROWS = 10240
ROWS_T = ACC_ROWS // NS

@functools.cache
def _mesh():
  return plsc.VectorSubcoreMesh(
      core_axis_name="c", subcore_axis_name="s", num_cores=NC,
      num_subcores=NS)

f32 = jnp.float32
i32 = jnp.int32
bf16 = jnp.bfloat16

_PI = tuple(list(range(0, LATENT, 2)) + list(range(1, LATENT, 2)))


_NSLOT = 4


def _sc_gather_sum(Ps, Pd, srcs, dsts):
  Ps = jax.lax.bitcast_convert_type(Ps.reshape(N, LATENT // 2, 2), i32)
  Pd = jax.lax.bitcast_convert_type(Pd.reshape(N, LATENT // 2, 2), i32)
  rows_w = srcs.shape[0] // NW
  ew = rows_w * CHUNK
  n_out = srcs.shape[0] * CHUNK
  srcs = srcs.reshape(NW, rows_w, CHUNK)
  dsts = dsts.reshape(NW, rows_w, CHUNK)

  @functools.partial(
      pl.kernel,
      out_type=jax.ShapeDtypeStruct((n_out, LATENT), i32),
      mesh=_mesh(),
      scratch_types=[
          pltpu.VMEM((rows_w, CHUNK), i32),
          pltpu.VMEM((rows_w, CHUNK), i32),
      ] + [pltpu.VMEM((CHUNK, LATENT // 2), i32)] * (2 * _NSLOT)
        + [pltpu.SemaphoreType.DMA] * (3 * _NSLOT),
      compiler_params=pltpu.CompilerParams(use_tc_tiling_on_sc=False),
  )
  def k(ps_hbm, pd_hbm, srcs_hbm, dsts_hbm, out_hbm, idx_s, idx_d, *rest):
    bufs_s = rest[0:_NSLOT]
    bufs_d = rest[_NSLOT:2 * _NSLOT]
    sems_s = rest[2 * _NSLOT:3 * _NSLOT]
    sems_d = rest[3 * _NSLOT:4 * _NSLOT]
    sems_w = rest[4 * _NSLOT:5 * _NSLOT]
    w = lax.axis_index("c") * NS + lax.axis_index("s")
    pltpu.sync_copy(srcs_hbm.at[w], idx_s)
    pltpu.sync_copy(dsts_hbm.at[w], idx_d)

    def g_issue(j, b):
      pltpu.async_copy(ps_hbm.at[idx_s.at[j]], bufs_s[b], sems_s[b])
      pltpu.async_copy(pd_hbm.at[idx_d.at[j]], bufs_d[b], sems_d[b])

    def g_wait(j, b):
      pltpu.make_async_copy(ps_hbm.at[idx_s.at[j]], bufs_s[b],
                            sems_s[b]).wait()
      pltpu.make_async_copy(pd_hbm.at[idx_d.at[j]], bufs_d[b],
                            sems_d[b]).wait()

    def out_s(j):
      return out_hbm.at[pl.ds(w * ew + j * CHUNK, CHUNK),
                        pl.ds(0, LATENT // 2)]

    def out_d(j):
      return out_hbm.at[pl.ds(w * ew + j * CHUNK, CHUNK),
                        pl.ds(LATENT // 2, LATENT // 2)]

    def w_issue(j, b):
      pltpu.async_copy(bufs_s[b], out_s(j), sems_w[b])
      pltpu.async_copy(bufs_d[b], out_d(j), sems_w[b])

    def w_drain(j, b):
      pltpu.make_async_copy(bufs_s[b], out_s(j), sems_w[b]).wait()
      pltpu.make_async_copy(bufs_d[b], out_d(j), sems_w[b]).wait()

    for b in range(_NSLOT):
      g_issue(b, b)
    n_grp = rows_w // _NSLOT

    def group_body(t, carry):
      for b in range(_NSLOT):
        j = _NSLOT * t + b
        g_wait(j, b)
        w_issue(j, b)
        w_drain(j, b)
        pl.when(t < n_grp - 1)(functools.partial(g_issue, j + _NSLOT, b))
      return carry

    lax.fori_loop(0, n_grp, group_body, 0)

  return k(Ps, Pd, srcs, dsts)


def _sc_feature_diff(T, srcs, dsts):
  D = 16

  @functools.partial(
      pl.kernel,
      out_type=jax.ShapeDtypeStruct((E_PAD, D), f32),
      mesh=_mesh(),
      scratch_types=[
          pltpu.VMEM((ROWS_W, CHUNK), i32),
          pltpu.VMEM((ROWS_W, CHUNK), i32),
          pltpu.VMEM((CHUNK, D), f32),
          pltpu.VMEM((CHUNK, D), f32),
          pltpu.VMEM((CHUNK, D), f32),
          pltpu.VMEM((CHUNK, D), f32),
          pltpu.SemaphoreType.DMA,
          pltpu.SemaphoreType.DMA,
          pltpu.SemaphoreType.DMA,
          pltpu.SemaphoreType.DMA,
          pltpu.SemaphoreType.DMA,
          pltpu.SemaphoreType.DMA,
      ],
      compiler_params=pltpu.CompilerParams(use_tc_tiling_on_sc=False),
  )
  def k(t_hbm, srcs_hbm, dsts_hbm, out_hbm, idx_s, idx_d,
        bs0, bd0, bs1, bd1, ss0, sd0, ss1, sd1, sw0, sw1):
    w = lax.axis_index("c") * NS + lax.axis_index("s")
    r0 = w * ROWS_W
    pltpu.sync_copy(srcs_hbm.at[pl.ds(r0, ROWS_W)], idx_s)
    pltpu.sync_copy(dsts_hbm.at[pl.ds(r0, ROWS_W)], idx_d)

    bufs = ((bs0, bd0, ss0, sd0, sw0), (bs1, bd1, ss1, sd1, sw1))

    def g_issue(j, slot):
      bs, bd, ss, sd, _ = bufs[slot]
      pltpu.async_copy(t_hbm.at[idx_s.at[j]], bs, ss)
      pltpu.async_copy(t_hbm.at[idx_d.at[j]], bd, sd)

    def g_wait(j, slot):
      bs, bd, ss, sd, _ = bufs[slot]
      pltpu.make_async_copy(t_hbm.at[idx_s.at[j]], bs, ss).wait()
      pltpu.make_async_copy(t_hbm.at[idx_d.at[j]], bd, sd).wait()

    def out_slice(j):
      return out_hbm.at[pl.ds(w * EW + j * CHUNK, CHUNK)]

    def sub_and_store(j, slot):
      bs, bd, _, _, sw = bufs[slot]

      def sub_row(r, c2):
        bs[r, pl.ds(0, 16)] = bs[r, pl.ds(0, 16)] - bd[r, pl.ds(0, 16)]
        return c2

      lax.fori_loop(0, CHUNK, sub_row, 0)
      pltpu.async_copy(bs, out_slice(j), sw)

    def w_drain(j, slot):
      bs, _, _, _, sw = bufs[slot]
      pltpu.make_async_copy(bs, out_slice(j), sw).wait()

    g_issue(0, 0)

    def pair_body(t, carry):
      j0 = 2 * t
      j1 = j0 + 1
      pl.when(t > 0)(lambda: w_drain(j1 - 2, 1))
      g_issue(j1, 1)
      g_wait(j0, 0)
      sub_and_store(j0, 0)
      w_drain(j0, 0)
      pl.when(t < ROWS_W // 2 - 1)(lambda: g_issue(j0 + 2, 0))
      g_wait(j1, 1)
      sub_and_store(j1, 1)
      return carry

    lax.fori_loop(0, ROWS_W // 2, pair_body, 0)
    w_drain(ROWS_W - 1, 1)

  return k(T, srcs, dsts)


def _sc_segment_sum(e_new, dsts):
  rows_w = dsts.shape[0] // NW
  ew = rows_w * CHUNK
  dsts = dsts.reshape(NW, rows_w, CHUNK)

  @functools.partial(
      pl.kernel,
      out_type=jax.ShapeDtypeStruct((NC, ACC_ROWS, LATENT), f32),
      mesh=_mesh(),
      scratch_types=[
          pltpu.VMEM((rows_w, CHUNK), i32),
          pltpu.VMEM((CHUNK, LATENT), f32),
          pltpu.VMEM((CHUNK, LATENT), f32),
          pltpu.VMEM_SHARED((ACC_ROWS, LATENT), f32),
          pltpu.SemaphoreType.DMA,
          pltpu.SemaphoreType.DMA,
      ],
  )
  def k(e_hbm, dsts_hbm, out_hbm, idx_d, eb0, eb1, acc, sr0, sr1):
    c = lax.axis_index("c")
    s = lax.axis_index("s")
    w = c * NS + s
    pltpu.sync_copy(dsts_hbm.at[w], idx_d)

    bufs = ((eb0, sr0), (eb1, sr1))

    def e_slice(j):
      return e_hbm.at[pl.ds(w * ew + j * CHUNK, CHUNK)]

    def r_issue(j, slot):
      eb, sr = bufs[slot]
      pltpu.async_copy(e_slice(j), eb, sr)

    def r_wait(j, slot):
      eb, sr = bufs[slot]
      pltpu.make_async_copy(e_slice(j), eb, sr).wait()

    def zero_row(r, carry):
      for q in range(LATENT // 16):
        eb0[r, pl.ds(q * 16, 16)] = jnp.zeros((16,), f32)
      return carry

    lax.fori_loop(0, CHUNK, zero_row, 0)
    for q in range(ROWS_T // CHUNK):
      pltpu.sync_copy(eb0, acc.at[pl.ds(s * ROWS_T + q * CHUNK, CHUNK)])
    plsc.subcore_barrier()

    r_issue(0, 0)

    def pair_body(t, carry):
      j0 = 2 * t
      j1 = j0 + 1
      r_issue(j1, 1)
      r_wait(j0, 0)
      pltpu.sync_copy(eb0, acc.at[idx_d.at[j0]], add=True)
      pl.when(t < rows_w // 2 - 1)(lambda: r_issue(j0 + 2, 0))
      r_wait(j1, 1)
      pltpu.sync_copy(eb1, acc.at[idx_d.at[j1]], add=True)
      return carry

    lax.fori_loop(0, rows_w // 2, pair_body, 0)
    plsc.subcore_barrier()

    for q in range(ROWS_T // CHUNK):
      r = s * ROWS_T + q * CHUNK
      pltpu.sync_copy(acc.at[pl.ds(r, CHUNK)], out_hbm.at[c, pl.ds(r, CHUNK)])

  return k(e_new, dsts)


BE = 2048
BN = 2000


def _ln(h, g, b):
  mu = jnp.mean(h, axis=-1, keepdims=True)
  xc = h - mu
  var = jnp.mean(xc * xc, axis=-1, keepdims=True)
  return xc * lax.rsqrt(var + 1e-5) * g + b


def _dot(a, b):
  return jnp.dot(a, b, preferred_element_type=f32)


def _full(shape):
  return pl.BlockSpec(shape, lambda i: tuple(0 for _ in shape))


def _tc_edge_encoder(diff, W0, W1, W2, consts, off, n_rows):
  def body(d_ref, w0_ref, w1_ref, w2_ref, c_ref, out_ref):
    d = d_ref[...]
    rw = d[:, 0:3]
    rm = d[:, 3:6]
    nw = jnp.sqrt(jnp.sum(rw * rw, axis=-1, keepdims=True) + 1e-12)
    nm = jnp.sqrt(jnp.sum(rm * rm, axis=-1, keepdims=True) + 1e-12)
    feat = jnp.concatenate([rw, nw, rm, nm], axis=-1)
    h0 = jnp.maximum(_dot(feat, w0_ref[...]) + c_ref[0], 0.0)
    h1 = jnp.maximum(_dot(h0, w1_ref[...]) + c_ref[1], 0.0)
    h2 = _dot(h1, w2_ref[...]) + c_ref[2]
    out_ref[...] = _ln(h2, c_ref[3], c_ref[4])

  return pl.pallas_call(
      body,
      grid=(n_rows // BE,),
      in_specs=[
          pl.BlockSpec((BE, 16), lambda i, o=off // BE: (i + o, 0)),
          _full((8, LATENT)),
          _full((LATENT, LATENT)),
          _full((LATENT, LATENT)),
          _full((8, LATENT)),
      ],
      out_specs=pl.BlockSpec((BE, LATENT), lambda i: (i, 0)),
      out_shape=jax.ShapeDtypeStruct((n_rows, LATENT), f32),
  )(diff, W0, W1, W2, consts)


def _tc_node_encoder(wp, pwp, tcol, W0, W1, W2, consts, Wp):
  def body(wp_ref, pwp_ref, t_ref, w0_ref, w1_ref, w2_ref, c_ref, wp_proj_ref,
           nlat_ref, ps_ref, pd_ref):
    vel = wp_ref[...] - pwp_ref[...]
    t = t_ref[...].astype(i32)
    oh = jnp.where(
        t == lax.broadcasted_iota(i32, (BN, NUM_TYPES), 1), 1.0, 0.0)
    feat = jnp.concatenate([vel, oh], axis=-1)
    h0 = jnp.maximum(_dot(feat, w0_ref[...]) + c_ref[0], 0.0)
    h1 = jnp.maximum(_dot(h0, w1_ref[...]) + c_ref[1], 0.0)
    h2 = _dot(h1, w2_ref[...]) + c_ref[2]
    nl = _ln(h2, c_ref[3], c_ref[4])
    nlat_ref[...] = nl
    proj = _dot(nl, wp_proj_ref[...])
    ps_ref[...] = proj[:, :LATENT].astype(bf16)
    pd_ref[...] = proj[:, LATENT:].astype(bf16)

  return pl.pallas_call(
      body,
      grid=(N // BN,),
      in_specs=[
          pl.BlockSpec((BN, 3), lambda i: (i, 0)),
          pl.BlockSpec((BN, 3), lambda i: (i, 0)),
          pl.BlockSpec((BN, 1), lambda i: (i, 0)),
          _full((NUM_TYPES + 3, LATENT)),
          _full((LATENT, LATENT)),
          _full((LATENT, LATENT)),
          _full((8, LATENT)),
          _full((LATENT, 2 * LATENT)),
      ],
      out_specs=[
          pl.BlockSpec((BN, LATENT), lambda i: (i, 0)),
          pl.BlockSpec((BN, LATENT), lambda i: (i, 0)),
          pl.BlockSpec((BN, LATENT), lambda i: (i, 0)),
      ],
      out_shape=[jax.ShapeDtypeStruct((N, LATENT), f32),
                 jax.ShapeDtypeStruct((N, LATENT), bf16),
                 jax.ShapeDtypeStruct((N, LATENT), bf16)],
  )(wp, pwp, tcol, W0, W1, W2, consts, Wp)


def _tc_edge_block(gsum, elat, W0e, W1, W2, consts):
  def body(g_ref, e_ref, w0_ref, w1_ref, w2_ref, c_ref, en_ref, eo_ref):
    e = e_ref[...]
    gi = g_ref[...]
    xs = gi[:, :LATENT // 2]
    xd = gi[:, LATENT // 2:]

    def lo(x):
      return jax.lax.bitcast_convert_type(x << 16, f32)

    def hi(x):
      return jax.lax.bitcast_convert_type(x & jnp.int32(-65536), f32)

    gp = jnp.concatenate([lo(xs) + lo(xd), hi(xs) + hi(xd)], axis=1)
    h0 = jnp.maximum(gp + _dot(e, w0_ref[...]) + c_ref[0], 0.0)
    h1 = jnp.maximum(_dot(h0, w1_ref[...]) + c_ref[1], 0.0)
    h2 = _dot(h1, w2_ref[...]) + c_ref[2]
    y = _ln(h2, c_ref[3], c_ref[4])
    en_ref[...] = y
    eo_ref[...] = e + y

  n_rows = gsum.shape[0]
  return pl.pallas_call(
      body,
      grid=(n_rows // BE,),
      in_specs=[
          pl.BlockSpec((BE, LATENT), lambda i: (i, 0)),
          pl.BlockSpec((BE, LATENT), lambda i: (i, 0)),
          _full((LATENT, LATENT)),
          _full((LATENT, LATENT)),
          _full((LATENT, LATENT)),
          _full((8, LATENT)),
      ],
      out_specs=[
          pl.BlockSpec((BE, LATENT), lambda i: (i, 0)),
          pl.BlockSpec((BE, LATENT), lambda i: (i, 0)),
      ],
      out_shape=[jax.ShapeDtypeStruct((n_rows, LATENT), f32)] * 2,
  )(gsum, elat, W0e, W1, W2, consts)


def _tc_node_block(nlat, agg2, agg2b, W0, W1, W2, consts, Wp):
  def body(n_ref, a_ref, ab_ref, w0_ref, w1_ref, w2_ref, c_ref, wp_ref,
           no_ref, ps_ref, pd_ref):
    nl = n_ref[...]
    agg = (a_ref[0] + a_ref[1]) + (ab_ref[0] + ab_ref[1])
    x = jnp.concatenate([nl, agg], axis=-1)
    h0 = jnp.maximum(_dot(x, w0_ref[...]) + c_ref[0], 0.0)
    h1 = jnp.maximum(_dot(h0, w1_ref[...]) + c_ref[1], 0.0)
    h2 = _dot(h1, w2_ref[...]) + c_ref[2]
    nl_new = nl + _ln(h2, c_ref[3], c_ref[4])
    no_ref[...] = nl_new
    proj = _dot(nl_new, wp_ref[...])
    ps_ref[...] = proj[:, :LATENT].astype(bf16)
    pd_ref[...] = proj[:, LATENT:].astype(bf16)

  return pl.pallas_call(
      body,
      grid=(N // BN,),
      in_specs=[
          pl.BlockSpec((BN, LATENT), lambda i: (i, 0)),
          pl.BlockSpec((NC, BN, LATENT), lambda i: (0, i, 0)),
          pl.BlockSpec((NC, BN, LATENT), lambda i: (0, i, 0)),
          _full((2 * LATENT, LATENT)),
          _full((LATENT, LATENT)),
          _full((LATENT, LATENT)),
          _full((8, LATENT)),
          _full((LATENT, 2 * LATENT)),
      ],
      out_specs=[
          pl.BlockSpec((BN, LATENT), lambda i: (i, 0)),
          pl.BlockSpec((BN, LATENT), lambda i: (i, 0)),
          pl.BlockSpec((BN, LATENT), lambda i: (i, 0)),
      ],
      out_shape=[jax.ShapeDtypeStruct((N, LATENT), f32),
                 jax.ShapeDtypeStruct((N, LATENT), bf16),
                 jax.ShapeDtypeStruct((N, LATENT), bf16)],
  )(nlat, agg2, agg2b, W0, W1, W2, consts, Wp)


def _tc_node_block_last(nlat, agg2, agg2b, W0, W1, W2, consts):
  def body(n_ref, a_ref, ab_ref, w0_ref, w1_ref, w2_ref, c_ref, no_ref):
    nl = n_ref[...]
    agg = (a_ref[0] + a_ref[1]) + (ab_ref[0] + ab_ref[1])
    x = jnp.concatenate([nl, agg], axis=-1)
    h0 = jnp.maximum(_dot(x, w0_ref[...]) + c_ref[0], 0.0)
    h1 = jnp.maximum(_dot(h0, w1_ref[...]) + c_ref[1], 0.0)
    h2 = _dot(h1, w2_ref[...]) + c_ref[2]
    no_ref[...] = nl + _ln(h2, c_ref[3], c_ref[4])

  return pl.pallas_call(
      body,
      grid=(N // BN,),
      in_specs=[
          pl.BlockSpec((BN, LATENT), lambda i: (i, 0)),
          pl.BlockSpec((NC, BN, LATENT), lambda i: (0, i, 0)),
          pl.BlockSpec((NC, BN, LATENT), lambda i: (0, i, 0)),
          _full((2 * LATENT, LATENT)),
          _full((LATENT, LATENT)),
          _full((LATENT, LATENT)),
          _full((8, LATENT)),
      ],
      out_specs=pl.BlockSpec((BN, LATENT), lambda i: (i, 0)),
      out_shape=jax.ShapeDtypeStruct((N, LATENT), f32),
  )(nlat, agg2, agg2b, W0, W1, W2, consts)


def _tc_decoder(nlat, wp, pwp, tcol, W0, W1, W2p, consts):
  def body(n_ref, wp_ref, pwp_ref, t_ref, w0_ref, w1_ref, w2_ref, c_ref,
           out_ref):
    h0 = jnp.maximum(_dot(n_ref[...], w0_ref[...]) + c_ref[0], 0.0)
    h1 = jnp.maximum(_dot(h0, w1_ref[...]) + c_ref[1], 0.0)
    h2 = _dot(h1, w2_ref[...]) + c_ref[2]
    acc = h2 * c_ref[3] + c_ref[4]
    wpv = wp_ref[...]
    pred_pos = 2.0 * wpv + acc[:, 0:3] - pwp_ref[...]
    mask = t_ref[...] == 0.0
    out_ref[...] = jnp.where(mask, pred_pos, wpv)

  return pl.pallas_call(
      body,
      grid=(N // BN,),
      in_specs=[
          pl.BlockSpec((BN, LATENT), lambda i: (i, 0)),
          pl.BlockSpec((BN, 3), lambda i: (i, 0)),
          pl.BlockSpec((BN, 3), lambda i: (i, 0)),
          pl.BlockSpec((BN, 1), lambda i: (i, 0)),
          _full((LATENT, LATENT)),
          _full((LATENT, LATENT)),
          _full((LATENT, LATENT)),
          _full((8, LATENT)),
      ],
      out_specs=pl.BlockSpec((BN, 3), lambda i: (i, 0)),
      out_shape=jax.ShapeDtypeStruct((N, 3), f32),
  )(nlat, wp, pwp, tcol, W0, W1, W2p, consts)


def _pack_consts(b0, b1, b2, g=None, b=None):
  rows = [b0, b1, b2]
  rows.append(g if g is not None else jnp.zeros((LATENT,), f32))
  rows.append(b if b is not None else jnp.zeros((LATENT,), f32))
  rows += [jnp.zeros((LATENT,), f32)] * 3
  return jnp.stack([jnp.pad(r, (0, LATENT - r.shape[0])) for r in rows])


def kernel(world_pos, prev_world_pos, mesh_pos, params, node_type, edge_index):
  src = edge_index[0].astype(i32)
  dst = edge_index[1].astype(i32)
  pad = E_PAD - E
  src_g = jnp.concatenate([src, jnp.zeros((pad,), i32)]).reshape(-1, CHUNK)
  dst_g = jnp.concatenate([dst, jnp.zeros((pad,), i32)]).reshape(-1, CHUNK)
  dst_s = jnp.concatenate([dst, jnp.full((pad,), N, i32)]).reshape(-1, CHUNK)

  T = jnp.concatenate(
      [world_pos, mesh_pos, jnp.zeros((N, 10), f32)], axis=1)
  tcol = node_type.astype(f32)[:, None]

  p = params

  def fold_first(mlp, mean, std):
    w0 = mlp['W0'] / std[:, None]
    b0 = mlp['b0'] - jnp.dot(mean / std, mlp['W0'])
    return w0, b0

  ew0, eb0 = fold_first(p['enc_edge'], p['mesh_norm']['mean'],
                        p['mesh_norm']['std'])
  nw0, nb0 = fold_first(p['enc_node'], p['node_norm']['mean'],
                        p['node_norm']['std'])
  enc_e = p['enc_edge']
  enc_n = p['enc_node']

  diff = _sc_feature_diff(T, src_g, dst_g)
  E_H = E_PAD // 2
  R_H = E_PAD // CHUNK // 2
  enc_consts = _pack_consts(eb0, enc_e['b1'], enc_e['b2'], enc_e['ln_g'],
                            enc_e['ln_b'])
  elat_a = _tc_edge_encoder(diff, ew0, enc_e['W1'], enc_e['W2'], enc_consts,
                            0, E_H)
  elat_b = _tc_edge_encoder(diff, ew0, enc_e['W1'], enc_e['W2'], enc_consts,
                            E_H, E_H)
  src_a, src_b = src_g[:R_H], src_g[R_H:]
  dst_a, dst_b = dst_g[:R_H], dst_g[R_H:]
  dsts_a, dsts_b = dst_s[:R_H], dst_s[R_H:]

  pi = jnp.array(_PI, i32)

  def proj_weights(blk):
    w0 = blk['edge_mlp']['W0']
    return jnp.concatenate([w0[:LATENT], w0[LATENT:2 * LATENT]], axis=1)

  nlat, Ps, Pd = _tc_node_encoder(
      world_pos, prev_world_pos, tcol, nw0, enc_n['W1'], enc_n['W2'],
      _pack_consts(nb0, enc_n['b1'], enc_n['b2'], enc_n['ln_g'],
                   enc_n['ln_b']),
      proj_weights(p['blocks'][0]))

  for b in range(len(p['blocks'])):
    blk = p['blocks'][b]
    em = blk['edge_mlp']
    nm = blk['node_mlp']
    econsts = _pack_consts(em['b0'][pi], em['b1'], em['b2'], em['ln_g'],
                           em['ln_b'])
    ew0e = em['W0'][2 * LATENT:][:, pi]
    ew1 = em['W1'][pi, :]
    gsum_a = _sc_gather_sum(Ps, Pd, src_a, dst_a)
    e_new_a, elat_a = _tc_edge_block(gsum_a, elat_a, ew0e, ew1, em['W2'],
                                     econsts)
    gsum_b = _sc_gather_sum(Ps, Pd, src_b, dst_b)
    agg_a = _sc_segment_sum(e_new_a, dsts_a)
    e_new_b, elat_b = _tc_edge_block(gsum_b, elat_b, ew0e, ew1, em['W2'],
                                     econsts)
    agg_b = _sc_segment_sum(e_new_b, dsts_b)
    nconsts = _pack_consts(nm['b0'], nm['b1'], nm['b2'], nm['ln_g'],
                           nm['ln_b'])
    if b + 1 < len(p['blocks']):
      nlat, Ps, Pd = _tc_node_block(
          nlat, agg_a, agg_b, nm['W0'], nm['W1'], nm['W2'], nconsts,
          proj_weights(p['blocks'][b + 1]))
    else:
      nlat = _tc_node_block_last(
          nlat, agg_a, agg_b, nm['W0'], nm['W1'], nm['W2'], nconsts)

  dec = p['dec']
  W2p = jnp.pad(dec['W2'], ((0, 0), (0, LATENT - 3)))
  dconsts = _pack_consts(
      dec['b0'], dec['b1'], dec['b2'],
      jnp.pad(p['out_norm']['std'], (0, LATENT - 3), constant_values=1.0),
      jnp.pad(p['out_norm']['mean'], (0, LATENT - 3)))
  return _tc_decoder(nlat, world_pos, prev_world_pos, tcol,
                     dec['W0'], dec['W1'], W2p, dconsts)
